# Initial kernel scaffold; baseline (speedup 1.0000x reference)
import jax
import jax.numpy as jnp
from jax.experimental import pallas as pl


def kernel(tr, mask, A_in_sta, A_in_src, W_init, b_init, W_l1t1, b_l1t1, W_l1t2, b_l1t2, W_l2t1a, b_l2t1a, W_l2t1b, b_l2t1b, W_l2t2a, b_l2t2a, W_l2t2b, b_l2t2b, a0, a11, a12, a1, a21, a22, a2):
    raise NotImplementedError("write your pallas kernel here")


# speedup vs baseline: 12.5620x; 12.5620x over previous
import functools

import jax
import jax.numpy as jnp
from jax import lax
from jax.experimental import pallas as pl
from jax.experimental.pallas import tpu as pltpu
from jax.experimental.pallas import tpu_sc as plsc

N = 100000
E = 1600000
F = 16
G = 128
CH = 4
NT = 16
GPT = 784
NCHUNK = GPT // CH
EPAD = NT * GPT * G
RPT = N // NT
ZR = 125
R = 1000


def _prelu(x, a):
    return jnp.where(x >= 0, x, a * x)


def _agg_body(tlo, thi, src_g, dst_g, out, src_v, dst_v, rows_v, zbuf, acc, sem):
    s = lax.axis_index("s")
    c = lax.axis_index("c")

    def run(table, plane):
        def zfill(i, _):
            zbuf[i] = jnp.zeros((F,), jnp.float32)
            return 0
        lax.fori_loop(0, ZR, zfill, 0)

        def zcopy(k, _):
            pltpu.sync_copy(zbuf, acc.at[pl.ds(s * RPT + k * ZR, ZR)])
            return 0
        lax.fori_loop(0, RPT // ZR, zcopy, 0)
        plsc.subcore_barrier()

        def load_fire(t):
            b = lax.rem(t, 2)
            g0 = s * GPT + t * CH
            pltpu.sync_copy(src_g.at[pl.ds(g0, CH)], src_v.at[b])
            pltpu.sync_copy(dst_g.at[pl.ds(g0, CH)], dst_v.at[b])

            def fire(j, _):
                pltpu.async_copy(table.at[src_v.at[b, j]], rows_v.at[b, j],
                                 sem.at[b])
                return 0
            lax.fori_loop(0, CH, fire, 0)

        def chunk(t, _):
            b = lax.rem(t, 2)

            @pl.when(t + 1 < NCHUNK)
            def _():
                load_fire(t + 1)

            def drain(j, _):
                pltpu.make_async_copy(table.at[src_v.at[b, j]],
                                      rows_v.at[b, j], sem.at[b]).wait()
                return 0
            lax.fori_loop(0, CH, drain, 0)

            def scat(j, _):
                pltpu.sync_copy(rows_v.at[b, j], acc.at[dst_v.at[b, j]],
                                add=True)
                return 0
            lax.fori_loop(0, CH, scat, 0)
            return 0

        load_fire(0)
        lax.fori_loop(0, NCHUNK, chunk, 0)
        plsc.subcore_barrier()

        pltpu.sync_copy(acc.at[pl.ds(s * RPT, RPT)], out.at[plane, s])

    @pl.when(c == 0)
    def _():
        run(tlo, 0)

    @pl.when(c == 1)
    def _():
        run(thi, 1)


def _sc_mean_agg(tlo, thi, src_g, dst_g):
    kern = pl.kernel(
        _agg_body,
        out_type=jax.ShapeDtypeStruct((2, NT, RPT, F), jnp.float32),
        mesh=plsc.VectorSubcoreMesh(core_axis_name="c", subcore_axis_name="s"),
        scratch_types=[
            pltpu.VMEM((2, CH, G), jnp.int32),
            pltpu.VMEM((2, CH, G), jnp.int32),
            pltpu.VMEM((2, CH, G, F), jnp.float32),
            pltpu.VMEM((ZR, F), jnp.float32),
            pltpu.VMEM_SHARED((N + 8, F), jnp.float32),
            pltpu.SemaphoreType.DMA((2,)),
        ],
        compiler_params=pltpu.CompilerParams(use_tc_tiling_on_sc=False),
    )
    return kern(tlo, thi, src_g, dst_g).reshape(2, N, F)


def _tc1_body(tr, msk, wt, wm, bias, av, h0o, m1lo, m1hi, m2lo, m2hi):
    a0 = av[0, 0]
    a11 = av[0, 1]
    a12 = av[0, 2]
    x = (jnp.dot(tr[...], wt[...], preferred_element_type=jnp.float32)
         + jnp.dot(msk[...], wm[...], preferred_element_type=jnp.float32)
         + bias[...])
    h0 = _prelu(x, a0)
    h0o[...] = h0
    m1 = _prelu(h0, a11)
    m2 = _prelu(h0, a12)
    lane = lax.broadcasted_iota(jnp.int32, (R, F), 1)
    one = lane == 14
    m1lo[...] = m1[:, :F]
    m1hi[...] = jnp.where(one, 1.0, m1[:, F:])
    m2lo[...] = m2[:, :F]
    m2hi[...] = jnp.where(one, 1.0, m2[:, F:])


def _tc2_body(h0, p1, p2, msk, w1h, w1pa, w1pb, w1m, b1,
              w2h, w2pa, w2pb, w2m, b2, A1a, A1b, ba1, A2a, A2b, ba2, av,
              h1ao, h1bo, g1lo, g1hi, g2lo, g2hi):
    a1 = av[0, 0]
    a21 = av[0, 1]
    a22 = av[0, 2]
    dot = functools.partial(jnp.dot, preferred_element_type=jnp.float32)
    h = h0[...]
    m = msk[...]
    p1a = p1[0]
    p1b = p1[1]
    p2a = p2[0]
    p2b = p2[1]
    c1 = jnp.maximum(p1b[:, 14:15], 1.0)
    c2 = jnp.maximum(p2b[:, 14:15], 1.0)
    t1 = (dot(h, w1h[...]) + dot(p1a / c1, w1pa[...]) + dot(p1b / c1, w1pb[...])
          + dot(m, w1m[...]) + b1[...])
    t2 = (dot(h, w2h[...]) + dot(p2a / c2, w2pa[...]) + dot(p2b / c2, w2pb[...])
          + dot(m, w2m[...]) + b2[...])
    h1a = _prelu(t1, a1)
    h1b = _prelu(t2, a1)
    h1ao[...] = h1a
    h1bo[...] = h1b
    g1 = _prelu(dot(h1a, A1a[...]) + dot(h1b, A1b[...]) + ba1[...], a21)
    g2 = _prelu(dot(h1a, A2a[...]) + dot(h1b, A2b[...]) + ba2[...], a22)
    lane = lax.broadcasted_iota(jnp.int32, (R, F), 1)
    one = lane == 14
    g1lo[...] = g1[:, :F]
    g1hi[...] = jnp.where(one, 1.0, g1[:, F:])
    g2lo[...] = g2[:, :F]
    g2hi[...] = jnp.where(one, 1.0, g2[:, F:])


def _tc3_body(h1a, h1b, q1, q2, msk, U1a, U1b, V1a, V1b, M1, c1b,
              U2a, U2b, V2a, V2b, M2, c2b, av, outo):
    a2 = av[0, 0]
    dot = functools.partial(jnp.dot, preferred_element_type=jnp.float32)
    ha = h1a[...]
    hb = h1b[...]
    m = msk[...]
    q1a = q1[0]
    q1b = q1[1]
    q2a = q2[0]
    q2b = q2[1]
    d1 = jnp.maximum(q1b[:, 14:15], 1.0)
    d2 = jnp.maximum(q2b[:, 14:15], 1.0)
    o1 = (dot(ha, U1a[...]) + dot(hb, U1b[...]) + dot(q1a / d1, V1a[...])
          + dot(q1b / d1, V1b[...]) + dot(m, M1[...]) + c1b[...])
    o2 = (dot(ha, U2a[...]) + dot(hb, U2b[...]) + dot(q2a / d2, V2a[...])
          + dot(q2b / d2, V2b[...]) + dot(m, M2[...]) + c2b[...])
    outo[...] = _prelu(jnp.concatenate([o1, o2], axis=1), a2)


def _full(shape):
    return pl.BlockSpec(shape, lambda i: tuple(0 for _ in shape))


def _rows(w):
    return pl.BlockSpec((R, w), lambda i: (i, 0))


def _plane(w):
    return pl.BlockSpec((2, R, w), lambda i: (0, i, 0))


def _pad(w, rpad, cpad):
    return jnp.pad(w, ((0, rpad), (0, cpad)))


def kernel(tr, mask, A_in_sta, A_in_src, W_init, b_init, W_l1t1, b_l1t1,
           W_l1t2, b_l1t2, W_l2t1a, b_l2t1a, W_l2t1b, b_l2t1b, W_l2t2a,
           b_l2t2a, W_l2t2b, b_l2t2b, a0, a11, a12, a1, a21, a22, a2):
    f32 = jnp.float32

    def prep_edges(A):
        src = jnp.concatenate([A[0].astype(jnp.int32),
                               jnp.zeros((EPAD - E,), jnp.int32)])
        dst = jnp.concatenate([A[1].astype(jnp.int32),
                               jnp.full((EPAD - E,), N, jnp.int32)])
        return src.reshape(-1, G), dst.reshape(-1, G)

    sta_s, sta_d = prep_edges(A_in_sta)
    srl_s, srl_d = prep_edges(A_in_src)

    wt = _pad(W_init[:128], 0, 2)
    wm = _pad(W_init[128:], 0, 2)
    bi = _pad(b_init[None, :], 0, 2)
    av1 = jnp.stack([a0, a11, a12, a0])[None, :]

    h0, m1lo, m1hi, m2lo, m2hi = pl.pallas_call(
        _tc1_body,
        grid=(N // R,),
        in_specs=[_rows(128), _rows(4), _full((128, 32)), _full((4, 32)),
                  _full((1, 32)), _full((1, 4))],
        out_specs=[_rows(32), _rows(F), _rows(F), _rows(F), _rows(F)],
        out_shape=[jax.ShapeDtypeStruct((N, 32), f32)]
        + [jax.ShapeDtypeStruct((N, F), f32)] * 4,
    )(tr, mask, wt, wm, bi, av1)

    p1 = _sc_mean_agg(m1lo, m1hi, sta_s, sta_d)
    p2 = _sc_mean_agg(m2lo, m2hi, srl_s, srl_d)

    def split1(Wf, bf):
        return (_pad(Wf[0:30], 2, 2), _pad(Wf[30:46], 0, 2),
                _pad(Wf[46:60], 2, 2), _pad(Wf[60:64], 0, 2),
                _pad(bf[None, :], 0, 2))
    w1h, w1pa, w1pb, w1m, b1 = split1(W_l1t1, b_l1t1)
    w2h, w2pa, w2pb, w2m, b2 = split1(W_l1t2, b_l1t2)
    A1a = _pad(W_l2t1a[0:30], 2, 2)
    A1b = _pad(W_l2t1a[30:60], 2, 2)
    ba1 = _pad(b_l2t1a[None, :], 0, 2)
    A2a = _pad(W_l2t2a[0:30], 2, 2)
    A2b = _pad(W_l2t2a[30:60], 2, 2)
    ba2 = _pad(b_l2t2a[None, :], 0, 2)
    av2 = jnp.stack([a1, a21, a22, a1])[None, :]

    h1a, h1b, g1lo, g1hi, g2lo, g2hi = pl.pallas_call(
        _tc2_body,
        grid=(N // R,),
        in_specs=[_rows(32), _plane(F), _plane(F), _rows(4),
                  _full((32, 32)), _full((16, 32)), _full((16, 32)),
                  _full((4, 32)), _full((1, 32)),
                  _full((32, 32)), _full((16, 32)), _full((16, 32)),
                  _full((4, 32)), _full((1, 32)),
                  _full((32, 32)), _full((32, 32)), _full((1, 32)),
                  _full((32, 32)), _full((32, 32)), _full((1, 32)),
                  _full((1, 4))],
        out_specs=[_rows(32), _rows(32), _rows(F), _rows(F), _rows(F),
                   _rows(F)],
        out_shape=[jax.ShapeDtypeStruct((N, 32), f32)] * 2
        + [jax.ShapeDtypeStruct((N, F), f32)] * 4,
    )(h0, p1, p2, mask, w1h, w1pa, w1pb, w1m, b1, w2h, w2pa, w2pb, w2m, b2,
      A1a, A1b, ba1, A2a, A2b, ba2, av2)

    q1 = _sc_mean_agg(g1lo, g1hi, sta_s, sta_d)
    q2 = _sc_mean_agg(g2lo, g2hi, srl_s, srl_d)

    def split3(Wf, bf):
        return (_pad(Wf[0:30], 2, 0), _pad(Wf[30:60], 2, 0), Wf[60:76],
                _pad(Wf[76:90], 2, 0), Wf[90:94], bf[None, :])
    U1a, U1b, V1a, V1b, M1, c1b = split3(W_l2t1b, b_l2t1b)
    U2a, U2b, V2a, V2b, M2, c2b = split3(W_l2t2b, b_l2t2b)
    av3 = jnp.stack([a2, a2, a2, a2])[None, :]

    out = pl.pallas_call(
        _tc3_body,
        grid=(N // R,),
        in_specs=[_rows(32), _rows(32), _plane(F), _plane(F), _rows(4),
                  _full((32, 128)), _full((32, 128)), _full((16, 128)),
                  _full((16, 128)), _full((4, 128)), _full((1, 128)),
                  _full((32, 128)), _full((32, 128)), _full((16, 128)),
                  _full((16, 128)), _full((4, 128)), _full((1, 128)),
                  _full((1, 4))],
        out_specs=[_rows(256)],
        out_shape=[jax.ShapeDtypeStruct((N, 256), f32)],
    )(h1a, h1b, q1, q2, mask, U1a, U1b, V1a, V1b, M1, c1b,
      U2a, U2b, V2a, V2b, M2, c2b, av3)

    return out[0]


# speedup vs baseline: 13.9182x; 1.1080x over previous
import functools

import jax
import jax.numpy as jnp
from jax import lax
from jax.experimental import pallas as pl
from jax.experimental.pallas import tpu as pltpu
from jax.experimental.pallas import tpu_sc as plsc

N = 100000
E = 1600000
F = 16
G = 128
CH = 4
NT = 16
GPT = 784
NCHUNK = GPT // CH
EPAD = NT * GPT * G
RPT = N // NT
ZR = 125
R = 1000


def _prelu(x, a):
    return jnp.where(x >= 0, x, a * x)


def _agg_body(tAlo, tAhi, tBlo, tBhi, sA, dA, sB, dB, outA, outB,
              src_v, dst_v, rows_v, zbuf, acc, gsem, ssem, isem, zsem):
    s = lax.axis_index("s")
    c = lax.axis_index("c")

    def zero():
        def zcopy(k, _):
            pltpu.async_copy(zbuf, acc.at[pl.ds(s * RPT + k * ZR, ZR)], zsem)
            return 0
        lax.fori_loop(0, RPT // ZR, zcopy, 0)

        def zdrain(k, _):
            pltpu.make_async_copy(
                zbuf, acc.at[pl.ds(s * RPT + k * ZR, ZR)], zsem).wait()
            return 0
        lax.fori_loop(0, RPT // ZR, zdrain, 0)

    def edges(table, src_g, dst_g):
        def load_idx(t, sync):
            sl = lax.rem(t, 3)
            b = lax.rem(t, 2)
            g0 = s * GPT + t * CH
            if sync:
                pltpu.sync_copy(src_g.at[pl.ds(g0, CH)], src_v.at[sl])
                pltpu.sync_copy(dst_g.at[pl.ds(g0, CH)], dst_v.at[sl])
            else:
                pltpu.async_copy(src_g.at[pl.ds(g0, CH)], src_v.at[sl],
                                 isem.at[b])
                pltpu.async_copy(dst_g.at[pl.ds(g0, CH)], dst_v.at[sl],
                                 isem.at[b])

        def wait_idx(t):
            sl = lax.rem(t, 3)
            b = lax.rem(t, 2)
            g0 = s * GPT + t * CH
            pltpu.make_async_copy(src_g.at[pl.ds(g0, CH)], src_v.at[sl],
                                  isem.at[b]).wait()
            pltpu.make_async_copy(dst_g.at[pl.ds(g0, CH)], dst_v.at[sl],
                                  isem.at[b]).wait()

        def fire(t):
            sl = lax.rem(t, 3)
            b = lax.rem(t, 2)

            def go(j, _):
                pltpu.async_copy(table.at[src_v.at[sl, j]], rows_v.at[b, j],
                                 gsem.at[b])
                return 0
            lax.fori_loop(0, CH, go, 0)

        def drain_g(t):
            sl = lax.rem(t, 3)
            b = lax.rem(t, 2)

            def go(j, _):
                pltpu.make_async_copy(table.at[src_v.at[sl, j]],
                                      rows_v.at[b, j], gsem.at[b]).wait()
                return 0
            lax.fori_loop(0, CH, go, 0)

        def scat(t):
            sl = lax.rem(t, 3)
            b = lax.rem(t, 2)

            def go(j, _):
                pltpu.async_copy(rows_v.at[b, j], acc.at[dst_v.at[sl, j]],
                                 ssem.at[b], add=True)
                return 0
            lax.fori_loop(0, CH, go, 0)

        def drain_s(t):
            sl = lax.rem(t, 3)
            b = lax.rem(t, 2)

            def go(j, _):
                pltpu.make_async_copy(rows_v.at[b, j], acc.at[dst_v.at[sl, j]],
                                      ssem.at[b]).wait()
                return 0
            lax.fori_loop(0, CH, go, 0)

        load_idx(0, True)
        fire(0)
        load_idx(1, True)

        def body(t, _):
            @pl.when(t >= 1)
            def _():
                drain_s(t - 1)

            @pl.when(t + 1 < NCHUNK)
            def _():
                @pl.when(t >= 1)
                def _():
                    wait_idx(t + 1)
                fire(t + 1)

            drain_g(t)
            scat(t)

            @pl.when(t + 2 < NCHUNK)
            def _():
                load_idx(t + 2, False)
            return 0

        lax.fori_loop(0, NCHUNK, body, 0)
        drain_s(NCHUNK - 1)

    def run(tA, tB, plane):
        def zfill(i, _):
            zbuf[i] = jnp.zeros((F,), jnp.float32)
            return 0
        lax.fori_loop(0, ZR, zfill, 0)

        zero()
        plsc.subcore_barrier()
        edges(tA, sA, dA)
        plsc.subcore_barrier()
        pltpu.sync_copy(acc.at[pl.ds(s * RPT, RPT)], outA.at[plane, s])
        zero()
        plsc.subcore_barrier()
        edges(tB, sB, dB)
        plsc.subcore_barrier()
        pltpu.sync_copy(acc.at[pl.ds(s * RPT, RPT)], outB.at[plane, s])

    @pl.when(c == 0)
    def _():
        run(tAlo, tBlo, 0)

    @pl.when(c == 1)
    def _():
        run(tAhi, tBhi, 1)


def _sc_mean_agg2(tAlo, tAhi, tBlo, tBhi, sA, dA, sB, dB):
    kern = pl.kernel(
        _agg_body,
        out_type=(jax.ShapeDtypeStruct((2, NT, RPT, F), jnp.float32),
                  jax.ShapeDtypeStruct((2, NT, RPT, F), jnp.float32)),
        mesh=plsc.VectorSubcoreMesh(core_axis_name="c", subcore_axis_name="s"),
        scratch_types=[
            pltpu.VMEM((3, CH, G), jnp.int32),
            pltpu.VMEM((3, CH, G), jnp.int32),
            pltpu.VMEM((2, CH, G, F), jnp.float32),
            pltpu.VMEM((ZR, F), jnp.float32),
            pltpu.VMEM_SHARED((N + 8, F), jnp.float32),
            pltpu.SemaphoreType.DMA((2,)),
            pltpu.SemaphoreType.DMA((2,)),
            pltpu.SemaphoreType.DMA((2,)),
            pltpu.SemaphoreType.DMA,
        ],
        compiler_params=pltpu.CompilerParams(use_tc_tiling_on_sc=False),
    )
    oA, oB = kern(tAlo, tAhi, tBlo, tBhi, sA, dA, sB, dB)
    return oA.reshape(2, N, F), oB.reshape(2, N, F)


def _tc1_body(tr, msk, wt, wm, bias, av, h0o, m1lo, m1hi, m2lo, m2hi):
    a0 = av[0, 0]
    a11 = av[0, 1]
    a12 = av[0, 2]
    x = (jnp.dot(tr[...], wt[...], preferred_element_type=jnp.float32)
         + jnp.dot(msk[...], wm[...], preferred_element_type=jnp.float32)
         + bias[...])
    h0 = _prelu(x, a0)
    h0o[...] = h0
    m1 = _prelu(h0, a11)
    m2 = _prelu(h0, a12)
    lane = lax.broadcasted_iota(jnp.int32, (R, F), 1)
    one = lane == 14
    m1lo[...] = m1[:, :F]
    m1hi[...] = jnp.where(one, 1.0, m1[:, F:])
    m2lo[...] = m2[:, :F]
    m2hi[...] = jnp.where(one, 1.0, m2[:, F:])


def _tc2_body(h0, p1, p2, msk, w1h, w1pa, w1pb, w1m, b1,
              w2h, w2pa, w2pb, w2m, b2, A1a, A1b, ba1, A2a, A2b, ba2, av,
              h1ao, h1bo, g1lo, g1hi, g2lo, g2hi):
    a1 = av[0, 0]
    a21 = av[0, 1]
    a22 = av[0, 2]
    dot = functools.partial(jnp.dot, preferred_element_type=jnp.float32)
    h = h0[...]
    m = msk[...]
    p1a = p1[0]
    p1b = p1[1]
    p2a = p2[0]
    p2b = p2[1]
    c1 = jnp.maximum(p1b[:, 14:15], 1.0)
    c2 = jnp.maximum(p2b[:, 14:15], 1.0)
    t1 = (dot(h, w1h[...]) + dot(p1a / c1, w1pa[...]) + dot(p1b / c1, w1pb[...])
          + dot(m, w1m[...]) + b1[...])
    t2 = (dot(h, w2h[...]) + dot(p2a / c2, w2pa[...]) + dot(p2b / c2, w2pb[...])
          + dot(m, w2m[...]) + b2[...])
    h1a = _prelu(t1, a1)
    h1b = _prelu(t2, a1)
    h1ao[...] = h1a
    h1bo[...] = h1b
    g1 = _prelu(dot(h1a, A1a[...]) + dot(h1b, A1b[...]) + ba1[...], a21)
    g2 = _prelu(dot(h1a, A2a[...]) + dot(h1b, A2b[...]) + ba2[...], a22)
    lane = lax.broadcasted_iota(jnp.int32, (R, F), 1)
    one = lane == 14
    g1lo[...] = g1[:, :F]
    g1hi[...] = jnp.where(one, 1.0, g1[:, F:])
    g2lo[...] = g2[:, :F]
    g2hi[...] = jnp.where(one, 1.0, g2[:, F:])


def _tc3_body(h1a, h1b, q1, q2, msk, U1a, U1b, V1a, V1b, M1, c1b,
              U2a, U2b, V2a, V2b, M2, c2b, av, outo):
    a2 = av[0, 0]
    dot = functools.partial(jnp.dot, preferred_element_type=jnp.float32)
    ha = h1a[...]
    hb = h1b[...]
    m = msk[...]
    q1a = q1[0]
    q1b = q1[1]
    q2a = q2[0]
    q2b = q2[1]
    d1 = jnp.maximum(q1b[:, 14:15], 1.0)
    d2 = jnp.maximum(q2b[:, 14:15], 1.0)
    o1 = (dot(ha, U1a[...]) + dot(hb, U1b[...]) + dot(q1a / d1, V1a[...])
          + dot(q1b / d1, V1b[...]) + dot(m, M1[...]) + c1b[...])
    o2 = (dot(ha, U2a[...]) + dot(hb, U2b[...]) + dot(q2a / d2, V2a[...])
          + dot(q2b / d2, V2b[...]) + dot(m, M2[...]) + c2b[...])
    outo[...] = _prelu(jnp.concatenate([o1, o2], axis=1), a2)


def _full(shape):
    return pl.BlockSpec(shape, lambda i: tuple(0 for _ in shape))


def _rows(w):
    return pl.BlockSpec((R, w), lambda i: (i, 0))


def _plane(w):
    return pl.BlockSpec((2, R, w), lambda i: (0, i, 0))


def _pad(w, rpad, cpad):
    return jnp.pad(w, ((0, rpad), (0, cpad)))


def kernel(tr, mask, A_in_sta, A_in_src, W_init, b_init, W_l1t1, b_l1t1,
           W_l1t2, b_l1t2, W_l2t1a, b_l2t1a, W_l2t1b, b_l2t1b, W_l2t2a,
           b_l2t2a, W_l2t2b, b_l2t2b, a0, a11, a12, a1, a21, a22, a2):
    f32 = jnp.float32

    def prep_edges(A):
        src = jnp.concatenate([A[0].astype(jnp.int32),
                               jnp.zeros((EPAD - E,), jnp.int32)])
        dst = jnp.concatenate([A[1].astype(jnp.int32),
                               jnp.full((EPAD - E,), N, jnp.int32)])
        return src.reshape(-1, G), dst.reshape(-1, G)

    sta_s, sta_d = prep_edges(A_in_sta)
    srl_s, srl_d = prep_edges(A_in_src)

    wt = _pad(W_init[:128], 0, 2)
    wm = _pad(W_init[128:], 0, 2)
    bi = _pad(b_init[None, :], 0, 2)
    av1 = jnp.stack([a0, a11, a12, a0])[None, :]

    h0, m1lo, m1hi, m2lo, m2hi = pl.pallas_call(
        _tc1_body,
        grid=(N // R,),
        in_specs=[_rows(128), _rows(4), _full((128, 32)), _full((4, 32)),
                  _full((1, 32)), _full((1, 4))],
        out_specs=[_rows(32), _rows(F), _rows(F), _rows(F), _rows(F)],
        out_shape=[jax.ShapeDtypeStruct((N, 32), f32)]
        + [jax.ShapeDtypeStruct((N, F), f32)] * 4,
    )(tr, mask, wt, wm, bi, av1)

    p1, p2 = _sc_mean_agg2(m1lo, m1hi, m2lo, m2hi, sta_s, sta_d, srl_s, srl_d)

    def split1(Wf, bf):
        return (_pad(Wf[0:30], 2, 2), _pad(Wf[30:46], 0, 2),
                _pad(Wf[46:60], 2, 2), _pad(Wf[60:64], 0, 2),
                _pad(bf[None, :], 0, 2))
    w1h, w1pa, w1pb, w1m, b1 = split1(W_l1t1, b_l1t1)
    w2h, w2pa, w2pb, w2m, b2 = split1(W_l1t2, b_l1t2)
    A1a = _pad(W_l2t1a[0:30], 2, 2)
    A1b = _pad(W_l2t1a[30:60], 2, 2)
    ba1 = _pad(b_l2t1a[None, :], 0, 2)
    A2a = _pad(W_l2t2a[0:30], 2, 2)
    A2b = _pad(W_l2t2a[30:60], 2, 2)
    ba2 = _pad(b_l2t2a[None, :], 0, 2)
    av2 = jnp.stack([a1, a21, a22, a1])[None, :]

    h1a, h1b, g1lo, g1hi, g2lo, g2hi = pl.pallas_call(
        _tc2_body,
        grid=(N // R,),
        in_specs=[_rows(32), _plane(F), _plane(F), _rows(4),
                  _full((32, 32)), _full((16, 32)), _full((16, 32)),
                  _full((4, 32)), _full((1, 32)),
                  _full((32, 32)), _full((16, 32)), _full((16, 32)),
                  _full((4, 32)), _full((1, 32)),
                  _full((32, 32)), _full((32, 32)), _full((1, 32)),
                  _full((32, 32)), _full((32, 32)), _full((1, 32)),
                  _full((1, 4))],
        out_specs=[_rows(32), _rows(32), _rows(F), _rows(F), _rows(F),
                   _rows(F)],
        out_shape=[jax.ShapeDtypeStruct((N, 32), f32)] * 2
        + [jax.ShapeDtypeStruct((N, F), f32)] * 4,
    )(h0, p1, p2, mask, w1h, w1pa, w1pb, w1m, b1, w2h, w2pa, w2pb, w2m, b2,
      A1a, A1b, ba1, A2a, A2b, ba2, av2)

    q1, q2 = _sc_mean_agg2(g1lo, g1hi, g2lo, g2hi, sta_s, sta_d, srl_s, srl_d)

    def split3(Wf, bf):
        return (_pad(Wf[0:30], 2, 0), _pad(Wf[30:60], 2, 0), Wf[60:76],
                _pad(Wf[76:90], 2, 0), Wf[90:94], bf[None, :])
    U1a, U1b, V1a, V1b, M1, c1b = split3(W_l2t1b, b_l2t1b)
    U2a, U2b, V2a, V2b, M2, c2b = split3(W_l2t2b, b_l2t2b)
    av3 = jnp.stack([a2, a2, a2, a2])[None, :]

    out = pl.pallas_call(
        _tc3_body,
        grid=(N // R,),
        in_specs=[_rows(32), _rows(32), _plane(F), _plane(F), _rows(4),
                  _full((32, 128)), _full((32, 128)), _full((16, 128)),
                  _full((16, 128)), _full((4, 128)), _full((1, 128)),
                  _full((32, 128)), _full((32, 128)), _full((16, 128)),
                  _full((16, 128)), _full((4, 128)), _full((1, 128)),
                  _full((1, 4))],
        out_specs=[_rows(256)],
        out_shape=[jax.ShapeDtypeStruct((N, 256), f32)],
    )(h1a, h1b, q1, q2, mask, U1a, U1b, V1a, V1b, M1, c1b,
      U2a, U2b, V2a, V2b, M2, c2b, av3)

    return out[0]


# speedup vs baseline: 14.4042x; 1.0349x over previous
import functools

import jax
import jax.numpy as jnp
from jax import lax
from jax.experimental import pallas as pl
from jax.experimental.pallas import tpu as pltpu
from jax.experimental.pallas import tpu_sc as plsc

N = 100000
E = 1600000
F = 16
G = 128
CH = 4
NT = 16
GPT = 784
NCHUNK = GPT // CH
NG0 = E // G
NGP = NT * GPT
PK = 8
RPT = N // NT
ZR = 125
R = 1000


def _prelu(x, a):
    return jnp.where(x >= 0, x, a * x)


def _agg_body(tAlo, tAhi, tBlo, tBhi, eA, eB, outA, outB,
              src_v, dst_v, rows_v, zbuf, acc, gsem, ssem, isem, zsem):
    s = lax.axis_index("s")
    c = lax.axis_index("c")
    sA, dA = eA.at[0], eA.at[1]
    sB, dB = eB.at[0], eB.at[1]

    def zero():
        def zcopy(k, _):
            pltpu.async_copy(zbuf, acc.at[pl.ds(s * RPT + k * ZR, ZR)], zsem)
            return 0
        lax.fori_loop(0, RPT // ZR, zcopy, 0)

        def zdrain(k, _):
            pltpu.make_async_copy(
                zbuf, acc.at[pl.ds(s * RPT + k * ZR, ZR)], zsem).wait()
            return 0
        lax.fori_loop(0, RPT // ZR, zdrain, 0)

    def edges(table, src_g, dst_g):
        def load_idx(t, sync):
            sl = lax.rem(t, 3)
            b = lax.rem(t, 2)
            g0 = s * GPT + t * CH
            if sync:
                pltpu.sync_copy(src_g.at[pl.ds(g0, CH)], src_v.at[sl])
                pltpu.sync_copy(dst_g.at[pl.ds(g0, CH)], dst_v.at[sl])
            else:
                pltpu.async_copy(src_g.at[pl.ds(g0, CH)], src_v.at[sl],
                                 isem.at[b])
                pltpu.async_copy(dst_g.at[pl.ds(g0, CH)], dst_v.at[sl],
                                 isem.at[b])

        def wait_idx(t):
            sl = lax.rem(t, 3)
            b = lax.rem(t, 2)
            g0 = s * GPT + t * CH
            pltpu.make_async_copy(src_g.at[pl.ds(g0, CH)], src_v.at[sl],
                                  isem.at[b]).wait()
            pltpu.make_async_copy(dst_g.at[pl.ds(g0, CH)], dst_v.at[sl],
                                  isem.at[b]).wait()

        def fire(t):
            sl = lax.rem(t, 3)
            b = lax.rem(t, 2)

            def go(j, _):
                pltpu.async_copy(table.at[src_v.at[sl, j]], rows_v.at[b, j],
                                 gsem.at[b])
                return 0
            lax.fori_loop(0, CH, go, 0)

        def drain_g(t):
            sl = lax.rem(t, 3)
            b = lax.rem(t, 2)

            def go(j, _):
                pltpu.make_async_copy(table.at[src_v.at[sl, j]],
                                      rows_v.at[b, j], gsem.at[b]).wait()
                return 0
            lax.fori_loop(0, CH, go, 0)

        def scat(t):
            sl = lax.rem(t, 3)
            b = lax.rem(t, 2)

            def go(j, _):
                pltpu.async_copy(rows_v.at[b, j], acc.at[dst_v.at[sl, j]],
                                 ssem.at[b], add=True)
                return 0
            lax.fori_loop(0, CH, go, 0)

        def drain_s(t):
            sl = lax.rem(t, 3)
            b = lax.rem(t, 2)

            def go(j, _):
                pltpu.make_async_copy(rows_v.at[b, j], acc.at[dst_v.at[sl, j]],
                                      ssem.at[b]).wait()
                return 0
            lax.fori_loop(0, CH, go, 0)

        load_idx(0, True)
        fire(0)
        load_idx(1, True)

        def body(t, _):
            @pl.when(t >= 1)
            def _():
                drain_s(t - 1)

            @pl.when(t + 1 < NCHUNK)
            def _():
                @pl.when(t >= 1)
                def _():
                    wait_idx(t + 1)
                fire(t + 1)

            drain_g(t)
            scat(t)

            @pl.when(t + 2 < NCHUNK)
            def _():
                load_idx(t + 2, False)
            return 0

        lax.fori_loop(0, NCHUNK, body, 0)
        drain_s(NCHUNK - 1)

    def run(tA, tB, plane):
        def zfill(i, _):
            zbuf[i] = jnp.zeros((F,), jnp.float32)
            return 0
        lax.fori_loop(0, ZR, zfill, 0)

        zero()
        plsc.subcore_barrier()
        edges(tA, sA, dA)
        plsc.subcore_barrier()
        pltpu.sync_copy(acc.at[pl.ds(s * RPT, RPT)], outA.at[plane, s])
        zero()
        plsc.subcore_barrier()
        edges(tB, sB, dB)
        plsc.subcore_barrier()
        pltpu.sync_copy(acc.at[pl.ds(s * RPT, RPT)], outB.at[plane, s])

    @pl.when(c == 0)
    def _():
        run(tAlo, tBlo, 0)

    @pl.when(c == 1)
    def _():
        run(tAhi, tBhi, 1)


def _sc_mean_agg2(tAlo, tAhi, tBlo, tBhi, eA, eB):
    kern = pl.kernel(
        _agg_body,
        out_type=(jax.ShapeDtypeStruct((2, NT, RPT, F), jnp.float32),
                  jax.ShapeDtypeStruct((2, NT, RPT, F), jnp.float32)),
        mesh=plsc.VectorSubcoreMesh(core_axis_name="c", subcore_axis_name="s"),
        scratch_types=[
            pltpu.VMEM((3, CH, G), jnp.int32),
            pltpu.VMEM((3, CH, G), jnp.int32),
            pltpu.VMEM((2, CH, G, F), jnp.float32),
            pltpu.VMEM((ZR, F), jnp.float32),
            pltpu.VMEM_SHARED((N + 8, F), jnp.float32),
            pltpu.SemaphoreType.DMA((2,)),
            pltpu.SemaphoreType.DMA((2,)),
            pltpu.SemaphoreType.DMA((2,)),
            pltpu.SemaphoreType.DMA,
        ],
        compiler_params=pltpu.CompilerParams(use_tc_tiling_on_sc=False),
    )
    oA, oB = kern(tAlo, tAhi, tBlo, tBhi, eA, eB)
    return oA.reshape(2, N, F), oB.reshape(2, N, F)


def _tc1_body(tr, msk, wt, wm, bias, av, h0o, m1lo, m1hi, m2lo, m2hi):
    a0 = av[0, 0]
    a11 = av[0, 1]
    a12 = av[0, 2]
    x = (jnp.dot(tr[...], wt[...], preferred_element_type=jnp.float32)
         + jnp.dot(msk[...], wm[...], preferred_element_type=jnp.float32)
         + bias[...])
    h0 = _prelu(x, a0)
    h0o[...] = h0
    m1 = _prelu(h0, a11)
    m2 = _prelu(h0, a12)
    lane = lax.broadcasted_iota(jnp.int32, (R, F), 1)
    one = lane == 14
    m1lo[...] = m1[:, :F]
    m1hi[...] = jnp.where(one, 1.0, m1[:, F:])
    m2lo[...] = m2[:, :F]
    m2hi[...] = jnp.where(one, 1.0, m2[:, F:])


def _tc2_body(h0, p1, p2, msk, w1h, w1pa, w1pb, w1m, b1,
              w2h, w2pa, w2pb, w2m, b2, A1a, A1b, ba1, A2a, A2b, ba2, av,
              h1ao, h1bo, g1lo, g1hi, g2lo, g2hi):
    a1 = av[0, 0]
    a21 = av[0, 1]
    a22 = av[0, 2]
    dot = functools.partial(jnp.dot, preferred_element_type=jnp.float32)
    h = h0[...]
    m = msk[...]
    p1a = p1[0]
    p1b = p1[1]
    p2a = p2[0]
    p2b = p2[1]
    c1 = jnp.maximum(p1b[:, 14:15], 1.0)
    c2 = jnp.maximum(p2b[:, 14:15], 1.0)
    t1 = (dot(h, w1h[...]) + dot(p1a / c1, w1pa[...]) + dot(p1b / c1, w1pb[...])
          + dot(m, w1m[...]) + b1[...])
    t2 = (dot(h, w2h[...]) + dot(p2a / c2, w2pa[...]) + dot(p2b / c2, w2pb[...])
          + dot(m, w2m[...]) + b2[...])
    h1a = _prelu(t1, a1)
    h1b = _prelu(t2, a1)
    h1ao[...] = h1a
    h1bo[...] = h1b
    g1 = _prelu(dot(h1a, A1a[...]) + dot(h1b, A1b[...]) + ba1[...], a21)
    g2 = _prelu(dot(h1a, A2a[...]) + dot(h1b, A2b[...]) + ba2[...], a22)
    lane = lax.broadcasted_iota(jnp.int32, (R, F), 1)
    one = lane == 14
    g1lo[...] = g1[:, :F]
    g1hi[...] = jnp.where(one, 1.0, g1[:, F:])
    g2lo[...] = g2[:, :F]
    g2hi[...] = jnp.where(one, 1.0, g2[:, F:])


def _tc3_body(h1a, h1b, q1, q2, msk, U1a, U1b, V1a, V1b, M1, c1b,
              U2a, U2b, V2a, V2b, M2, c2b, av, outo):
    a2 = av[0, 0]
    dot = functools.partial(jnp.dot, preferred_element_type=jnp.float32)
    ha = h1a[...]
    hb = h1b[...]
    m = msk[...]
    q1a = q1[0]
    q1b = q1[1]
    q2a = q2[0]
    q2b = q2[1]
    d1 = jnp.maximum(q1b[:, 14:15], 1.0)
    d2 = jnp.maximum(q2b[:, 14:15], 1.0)
    o1 = (dot(ha, U1a[...]) + dot(hb, U1b[...]) + dot(q1a / d1, V1a[...])
          + dot(q1b / d1, V1b[...]) + dot(m, M1[...]) + c1b[...])
    o2 = (dot(ha, U2a[...]) + dot(hb, U2b[...]) + dot(q2a / d2, V2a[...])
          + dot(q2b / d2, V2b[...]) + dot(m, M2[...]) + c2b[...])
    outo[...] = _prelu(jnp.concatenate([o1, o2], axis=1), a2)


def _full(shape):
    return pl.BlockSpec(shape, lambda i: tuple(0 for _ in shape))


def _rows(w):
    return pl.BlockSpec((R, w), lambda i: (i, 0))


def _plane(w):
    return pl.BlockSpec((2, R, w), lambda i: (0, i, 0))


def _pad(w, rpad, cpad):
    return jnp.pad(w, ((0, rpad), (0, cpad)))


def kernel(tr, mask, A_in_sta, A_in_src, W_init, b_init, W_l1t1, b_l1t1,
           W_l1t2, b_l1t2, W_l2t1a, b_l2t1a, W_l2t1b, b_l2t1b, W_l2t2a,
           b_l2t2a, W_l2t2b, b_l2t2b, a0, a11, a12, a1, a21, a22, a2):
    f32 = jnp.float32

    pad_blk = jnp.stack([jnp.zeros((NGP - NG0, G), jnp.int32),
                         jnp.full((NGP - NG0, G), N, jnp.int32)])

    def prep_edges(A):
        return jnp.concatenate(
            [A.astype(jnp.int32).reshape(2, NG0, G), pad_blk], axis=1)

    eA = prep_edges(A_in_sta)
    eB = prep_edges(A_in_src)

    wt = _pad(W_init[:128], 0, 2)
    wm = _pad(W_init[128:], 0, 2)
    bi = _pad(b_init[None, :], 0, 2)
    av1 = jnp.stack([a0, a11, a12, a0])[None, :]

    h0, m1lo, m1hi, m2lo, m2hi = pl.pallas_call(
        _tc1_body,
        grid=(N // R,),
        in_specs=[_rows(128), _rows(4), _full((128, 32)), _full((4, 32)),
                  _full((1, 32)), _full((1, 4))],
        out_specs=[_rows(32), _rows(F), _rows(F), _rows(F), _rows(F)],
        out_shape=[jax.ShapeDtypeStruct((N, 32), f32)]
        + [jax.ShapeDtypeStruct((N, F), f32)] * 4,
    )(tr, mask, wt, wm, bi, av1)

    p1, p2 = _sc_mean_agg2(m1lo, m1hi, m2lo, m2hi, eA, eB)

    def split1(Wf, bf):
        return (_pad(Wf[0:30], 2, 2), _pad(Wf[30:46], 0, 2),
                _pad(Wf[46:60], 2, 2), _pad(Wf[60:64], 0, 2),
                _pad(bf[None, :], 0, 2))
    w1h, w1pa, w1pb, w1m, b1 = split1(W_l1t1, b_l1t1)
    w2h, w2pa, w2pb, w2m, b2 = split1(W_l1t2, b_l1t2)
    A1a = _pad(W_l2t1a[0:30], 2, 2)
    A1b = _pad(W_l2t1a[30:60], 2, 2)
    ba1 = _pad(b_l2t1a[None, :], 0, 2)
    A2a = _pad(W_l2t2a[0:30], 2, 2)
    A2b = _pad(W_l2t2a[30:60], 2, 2)
    ba2 = _pad(b_l2t2a[None, :], 0, 2)
    av2 = jnp.stack([a1, a21, a22, a1])[None, :]

    h1a, h1b, g1lo, g1hi, g2lo, g2hi = pl.pallas_call(
        _tc2_body,
        grid=(N // R,),
        in_specs=[_rows(32), _plane(F), _plane(F), _rows(4),
                  _full((32, 32)), _full((16, 32)), _full((16, 32)),
                  _full((4, 32)), _full((1, 32)),
                  _full((32, 32)), _full((16, 32)), _full((16, 32)),
                  _full((4, 32)), _full((1, 32)),
                  _full((32, 32)), _full((32, 32)), _full((1, 32)),
                  _full((32, 32)), _full((32, 32)), _full((1, 32)),
                  _full((1, 4))],
        out_specs=[_rows(32), _rows(32), _rows(F), _rows(F), _rows(F),
                   _rows(F)],
        out_shape=[jax.ShapeDtypeStruct((N, 32), f32)] * 2
        + [jax.ShapeDtypeStruct((N, F), f32)] * 4,
    )(h0, p1, p2, mask, w1h, w1pa, w1pb, w1m, b1, w2h, w2pa, w2pb, w2m, b2,
      A1a, A1b, ba1, A2a, A2b, ba2, av2)

    q1, q2 = _sc_mean_agg2(g1lo, g1hi, g2lo, g2hi, eA, eB)

    def split3(Wf, bf):
        return (_pad(Wf[0:30], 2, 0), _pad(Wf[30:60], 2, 0), Wf[60:76],
                _pad(Wf[76:90], 2, 0), Wf[90:94], bf[None, :])
    U1a, U1b, V1a, V1b, M1, c1b = split3(W_l2t1b, b_l2t1b)
    U2a, U2b, V2a, V2b, M2, c2b = split3(W_l2t2b, b_l2t2b)
    av3 = jnp.stack([a2, a2, a2, a2])[None, :]

    out = pl.pallas_call(
        _tc3_body,
        grid=(N // R,),
        in_specs=[_rows(32), _rows(32), _plane(F), _plane(F), _rows(4),
                  _full((32, 128)), _full((32, 128)), _full((16, 128)),
                  _full((16, 128)), _full((4, 128)), _full((1, 128)),
                  _full((32, 128)), _full((32, 128)), _full((16, 128)),
                  _full((16, 128)), _full((4, 128)), _full((1, 128)),
                  _full((1, 4))],
        out_specs=[_rows(256)],
        out_shape=[jax.ShapeDtypeStruct((N, 256), f32)],
    )(h1a, h1b, q1, q2, mask, U1a, U1b, V1a, V1b, M1, c1b,
      U2a, U2b, V2a, V2b, M2, c2b, av3)

    return out[0]


# speedup vs baseline: 14.4105x; 1.0004x over previous
import functools

import jax
import jax.numpy as jnp
from jax import lax
from jax.experimental import pallas as pl
from jax.experimental.pallas import tpu as pltpu
from jax.experimental.pallas import tpu_sc as plsc

N = 100000
E = 1600000
F = 16
G = 128
CH = 4
NT = 16
GPT = 784
NCHUNK = GPT // CH
NG0 = E // G
NGP = NT * GPT
PK = 8
RPT = N // NT
ZR = 125
R = 1000


def _prelu(x, a):
    return jnp.where(x >= 0, x, a * x)


def _agg_body(tAlo, tAhi, tBlo, tBhi, eA, eB, outA, outB,
              src_v, dst_v, rows_v, zbuf, acc, gsem, ssem, isem, zsem):
    s = lax.axis_index("s")
    c = lax.axis_index("c")
    sA, dA = eA.at[0], eA.at[1]
    sB, dB = eB.at[0], eB.at[1]
    CHG = CH * G

    def zero():
        def zcopy(k, _):
            pltpu.async_copy(zbuf, acc.at[pl.ds(s * RPT + k * ZR, ZR)], zsem)
            return 0
        lax.fori_loop(0, RPT // ZR, zcopy, 0)

        def zdrain(k, _):
            pltpu.make_async_copy(
                zbuf, acc.at[pl.ds(s * RPT + k * ZR, ZR)], zsem).wait()
            return 0
        lax.fori_loop(0, RPT // ZR, zdrain, 0)

    def edges(table, src_g, dst_g):
        def load_idx(t, sync):
            sl = lax.rem(t, 3)
            b = lax.rem(t, 2)
            e0 = (s * GPT + t * CH) * G
            if sync:
                pltpu.sync_copy(src_g.at[pl.ds(e0, CHG)], src_v.at[sl])
                pltpu.sync_copy(dst_g.at[pl.ds(e0, CHG)], dst_v.at[sl])
            else:
                pltpu.async_copy(src_g.at[pl.ds(e0, CHG)], src_v.at[sl],
                                 isem.at[b])
                pltpu.async_copy(dst_g.at[pl.ds(e0, CHG)], dst_v.at[sl],
                                 isem.at[b])

        def wait_idx(t):
            sl = lax.rem(t, 3)
            b = lax.rem(t, 2)
            e0 = (s * GPT + t * CH) * G
            pltpu.make_async_copy(src_g.at[pl.ds(e0, CHG)], src_v.at[sl],
                                  isem.at[b]).wait()
            pltpu.make_async_copy(dst_g.at[pl.ds(e0, CHG)], dst_v.at[sl],
                                  isem.at[b]).wait()

        def fire(t):
            sl = lax.rem(t, 3)
            b = lax.rem(t, 2)
            pltpu.async_copy(table.at[src_v.at[sl]], rows_v.at[b], gsem.at[b])

        def drain_g(t):
            sl = lax.rem(t, 3)
            b = lax.rem(t, 2)
            pltpu.make_async_copy(table.at[src_v.at[sl]], rows_v.at[b],
                                  gsem.at[b]).wait()

        def scat(t):
            sl = lax.rem(t, 3)
            b = lax.rem(t, 2)
            pltpu.async_copy(rows_v.at[b], acc.at[dst_v.at[sl]], ssem.at[b],
                             add=True)

        def drain_s(t):
            sl = lax.rem(t, 3)
            b = lax.rem(t, 2)
            pltpu.make_async_copy(rows_v.at[b], acc.at[dst_v.at[sl]],
                                  ssem.at[b]).wait()

        load_idx(0, True)
        fire(0)
        load_idx(1, True)

        def body(t, _):
            @pl.when(t >= 1)
            def _():
                drain_s(t - 1)

            @pl.when(t + 1 < NCHUNK)
            def _():
                @pl.when(t >= 1)
                def _():
                    wait_idx(t + 1)
                fire(t + 1)

            drain_g(t)
            scat(t)

            @pl.when(t + 2 < NCHUNK)
            def _():
                load_idx(t + 2, False)
            return 0

        lax.fori_loop(0, NCHUNK, body, 0)
        drain_s(NCHUNK - 1)

    def run(tA, tB, plane):
        def zfill(i, _):
            zbuf[i] = jnp.zeros((F,), jnp.float32)
            return 0
        lax.fori_loop(0, ZR, zfill, 0)

        zero()
        plsc.subcore_barrier()
        edges(tA, sA, dA)
        plsc.subcore_barrier()
        pltpu.sync_copy(acc.at[pl.ds(s * RPT, RPT)], outA.at[plane, s])
        zero()
        plsc.subcore_barrier()
        edges(tB, sB, dB)
        plsc.subcore_barrier()
        pltpu.sync_copy(acc.at[pl.ds(s * RPT, RPT)], outB.at[plane, s])

    @pl.when(c == 0)
    def _():
        run(tAlo, tBlo, 0)

    @pl.when(c == 1)
    def _():
        run(tAhi, tBhi, 1)


def _sc_mean_agg2(tAlo, tAhi, tBlo, tBhi, eA, eB):
    kern = pl.kernel(
        _agg_body,
        out_type=(jax.ShapeDtypeStruct((2, NT, RPT, F), jnp.float32),
                  jax.ShapeDtypeStruct((2, NT, RPT, F), jnp.float32)),
        mesh=plsc.VectorSubcoreMesh(core_axis_name="c", subcore_axis_name="s"),
        scratch_types=[
            pltpu.VMEM((3, CH * G), jnp.int32),
            pltpu.VMEM((3, CH * G), jnp.int32),
            pltpu.VMEM((2, CH * G, F), jnp.float32),
            pltpu.VMEM((ZR, F), jnp.float32),
            pltpu.VMEM_SHARED((N + 8, F), jnp.float32),
            pltpu.SemaphoreType.DMA((2,)),
            pltpu.SemaphoreType.DMA((2,)),
            pltpu.SemaphoreType.DMA((2,)),
            pltpu.SemaphoreType.DMA,
        ],
        compiler_params=pltpu.CompilerParams(use_tc_tiling_on_sc=False),
    )
    oA, oB = kern(tAlo, tAhi, tBlo, tBhi,
                  eA.reshape(2, NGP * G), eB.reshape(2, NGP * G))
    return oA.reshape(2, N, F), oB.reshape(2, N, F)


def _tc1_body(tr, msk, wt, wm, bias, av, h0o, m1lo, m1hi, m2lo, m2hi):
    a0 = av[0, 0]
    a11 = av[0, 1]
    a12 = av[0, 2]
    x = (jnp.dot(tr[...], wt[...], preferred_element_type=jnp.float32)
         + jnp.dot(msk[...], wm[...], preferred_element_type=jnp.float32)
         + bias[...])
    h0 = _prelu(x, a0)
    h0o[...] = h0
    m1 = _prelu(h0, a11)
    m2 = _prelu(h0, a12)
    lane = lax.broadcasted_iota(jnp.int32, (R, F), 1)
    one = lane == 14
    m1lo[...] = m1[:, :F]
    m1hi[...] = jnp.where(one, 1.0, m1[:, F:])
    m2lo[...] = m2[:, :F]
    m2hi[...] = jnp.where(one, 1.0, m2[:, F:])


def _tc2_body(h0, p1, p2, msk, w1h, w1pa, w1pb, w1m, b1,
              w2h, w2pa, w2pb, w2m, b2, A1a, A1b, ba1, A2a, A2b, ba2, av,
              h1ao, h1bo, g1lo, g1hi, g2lo, g2hi):
    a1 = av[0, 0]
    a21 = av[0, 1]
    a22 = av[0, 2]
    dot = functools.partial(jnp.dot, preferred_element_type=jnp.float32)
    h = h0[...]
    m = msk[...]
    p1a = p1[0]
    p1b = p1[1]
    p2a = p2[0]
    p2b = p2[1]
    c1 = jnp.maximum(p1b[:, 14:15], 1.0)
    c2 = jnp.maximum(p2b[:, 14:15], 1.0)
    t1 = (dot(h, w1h[...]) + dot(p1a / c1, w1pa[...]) + dot(p1b / c1, w1pb[...])
          + dot(m, w1m[...]) + b1[...])
    t2 = (dot(h, w2h[...]) + dot(p2a / c2, w2pa[...]) + dot(p2b / c2, w2pb[...])
          + dot(m, w2m[...]) + b2[...])
    h1a = _prelu(t1, a1)
    h1b = _prelu(t2, a1)
    h1ao[...] = h1a
    h1bo[...] = h1b
    g1 = _prelu(dot(h1a, A1a[...]) + dot(h1b, A1b[...]) + ba1[...], a21)
    g2 = _prelu(dot(h1a, A2a[...]) + dot(h1b, A2b[...]) + ba2[...], a22)
    lane = lax.broadcasted_iota(jnp.int32, (R, F), 1)
    one = lane == 14
    g1lo[...] = g1[:, :F]
    g1hi[...] = jnp.where(one, 1.0, g1[:, F:])
    g2lo[...] = g2[:, :F]
    g2hi[...] = jnp.where(one, 1.0, g2[:, F:])


def _tc3_body(h1a, h1b, q1, q2, msk, U1a, U1b, V1a, V1b, M1, c1b,
              U2a, U2b, V2a, V2b, M2, c2b, av, outo):
    a2 = av[0, 0]
    dot = functools.partial(jnp.dot, preferred_element_type=jnp.float32)
    ha = h1a[...]
    hb = h1b[...]
    m = msk[...]
    q1a = q1[0]
    q1b = q1[1]
    q2a = q2[0]
    q2b = q2[1]
    d1 = jnp.maximum(q1b[:, 14:15], 1.0)
    d2 = jnp.maximum(q2b[:, 14:15], 1.0)
    o1 = (dot(ha, U1a[...]) + dot(hb, U1b[...]) + dot(q1a / d1, V1a[...])
          + dot(q1b / d1, V1b[...]) + dot(m, M1[...]) + c1b[...])
    o2 = (dot(ha, U2a[...]) + dot(hb, U2b[...]) + dot(q2a / d2, V2a[...])
          + dot(q2b / d2, V2b[...]) + dot(m, M2[...]) + c2b[...])
    outo[...] = _prelu(jnp.concatenate([o1, o2], axis=1), a2)


def _full(shape):
    return pl.BlockSpec(shape, lambda i: tuple(0 for _ in shape))


def _rows(w):
    return pl.BlockSpec((R, w), lambda i: (i, 0))


def _plane(w):
    return pl.BlockSpec((2, R, w), lambda i: (0, i, 0))


def _pad(w, rpad, cpad):
    return jnp.pad(w, ((0, rpad), (0, cpad)))


def kernel(tr, mask, A_in_sta, A_in_src, W_init, b_init, W_l1t1, b_l1t1,
           W_l1t2, b_l1t2, W_l2t1a, b_l2t1a, W_l2t1b, b_l2t1b, W_l2t2a,
           b_l2t2a, W_l2t2b, b_l2t2b, a0, a11, a12, a1, a21, a22, a2):
    f32 = jnp.float32

    pad_blk = jnp.stack([jnp.zeros((NGP - NG0, G), jnp.int32),
                         jnp.full((NGP - NG0, G), N, jnp.int32)])

    def prep_edges(A):
        return jnp.concatenate(
            [A.astype(jnp.int32).reshape(2, NG0, G), pad_blk], axis=1)

    eA = prep_edges(A_in_sta)
    eB = prep_edges(A_in_src)

    wt = _pad(W_init[:128], 0, 2)
    wm = _pad(W_init[128:], 0, 2)
    bi = _pad(b_init[None, :], 0, 2)
    av1 = jnp.stack([a0, a11, a12, a0])[None, :]

    h0, m1lo, m1hi, m2lo, m2hi = pl.pallas_call(
        _tc1_body,
        grid=(N // R,),
        in_specs=[_rows(128), _rows(4), _full((128, 32)), _full((4, 32)),
                  _full((1, 32)), _full((1, 4))],
        out_specs=[_rows(32), _rows(F), _rows(F), _rows(F), _rows(F)],
        out_shape=[jax.ShapeDtypeStruct((N, 32), f32)]
        + [jax.ShapeDtypeStruct((N, F), f32)] * 4,
    )(tr, mask, wt, wm, bi, av1)

    p1, p2 = _sc_mean_agg2(m1lo, m1hi, m2lo, m2hi, eA, eB)

    def split1(Wf, bf):
        return (_pad(Wf[0:30], 2, 2), _pad(Wf[30:46], 0, 2),
                _pad(Wf[46:60], 2, 2), _pad(Wf[60:64], 0, 2),
                _pad(bf[None, :], 0, 2))
    w1h, w1pa, w1pb, w1m, b1 = split1(W_l1t1, b_l1t1)
    w2h, w2pa, w2pb, w2m, b2 = split1(W_l1t2, b_l1t2)
    A1a = _pad(W_l2t1a[0:30], 2, 2)
    A1b = _pad(W_l2t1a[30:60], 2, 2)
    ba1 = _pad(b_l2t1a[None, :], 0, 2)
    A2a = _pad(W_l2t2a[0:30], 2, 2)
    A2b = _pad(W_l2t2a[30:60], 2, 2)
    ba2 = _pad(b_l2t2a[None, :], 0, 2)
    av2 = jnp.stack([a1, a21, a22, a1])[None, :]

    h1a, h1b, g1lo, g1hi, g2lo, g2hi = pl.pallas_call(
        _tc2_body,
        grid=(N // R,),
        in_specs=[_rows(32), _plane(F), _plane(F), _rows(4),
                  _full((32, 32)), _full((16, 32)), _full((16, 32)),
                  _full((4, 32)), _full((1, 32)),
                  _full((32, 32)), _full((16, 32)), _full((16, 32)),
                  _full((4, 32)), _full((1, 32)),
                  _full((32, 32)), _full((32, 32)), _full((1, 32)),
                  _full((32, 32)), _full((32, 32)), _full((1, 32)),
                  _full((1, 4))],
        out_specs=[_rows(32), _rows(32), _rows(F), _rows(F), _rows(F),
                   _rows(F)],
        out_shape=[jax.ShapeDtypeStruct((N, 32), f32)] * 2
        + [jax.ShapeDtypeStruct((N, F), f32)] * 4,
    )(h0, p1, p2, mask, w1h, w1pa, w1pb, w1m, b1, w2h, w2pa, w2pb, w2m, b2,
      A1a, A1b, ba1, A2a, A2b, ba2, av2)

    q1, q2 = _sc_mean_agg2(g1lo, g1hi, g2lo, g2hi, eA, eB)

    def split3(Wf, bf):
        return (_pad(Wf[0:30], 2, 0), _pad(Wf[30:60], 2, 0), Wf[60:76],
                _pad(Wf[76:90], 2, 0), Wf[90:94], bf[None, :])
    U1a, U1b, V1a, V1b, M1, c1b = split3(W_l2t1b, b_l2t1b)
    U2a, U2b, V2a, V2b, M2, c2b = split3(W_l2t2b, b_l2t2b)
    av3 = jnp.stack([a2, a2, a2, a2])[None, :]

    out = pl.pallas_call(
        _tc3_body,
        grid=(N // R,),
        in_specs=[_rows(32), _rows(32), _plane(F), _plane(F), _rows(4),
                  _full((32, 128)), _full((32, 128)), _full((16, 128)),
                  _full((16, 128)), _full((4, 128)), _full((1, 128)),
                  _full((32, 128)), _full((32, 128)), _full((16, 128)),
                  _full((16, 128)), _full((4, 128)), _full((1, 128)),
                  _full((1, 4))],
        out_specs=[_rows(256)],
        out_shape=[jax.ShapeDtypeStruct((N, 256), f32)],
    )(h1a, h1b, q1, q2, mask, U1a, U1b, V1a, V1b, M1, c1b,
      U2a, U2b, V2a, V2b, M2, c2b, av3)

    return out[0]


# speedup vs baseline: 15.4019x; 1.0688x over previous
import functools

import jax
import jax.numpy as jnp
from jax import lax
from jax.experimental import pallas as pl
from jax.experimental.pallas import tpu as pltpu
from jax.experimental.pallas import tpu_sc as plsc

N = 100000
E = 1600000
F = 16
G = 128
CH = 4
NT = 16
GPT = 784
NCHUNK = GPT // CH
NG0 = E // G
NGP = NT * GPT
PK = 8
RPT = N // NT
ZR = 125
R = 2000


def _prelu(x, a):
    return jnp.where(x >= 0, x, a * x)


def _agg_body(tAlo, tAhi, tBlo, tBhi, eA, eB, outA, outB,
              src_v, dst_v, rows_v, zbuf, acc, gsem, ssem, isem, zsem):
    s = lax.axis_index("s")
    c = lax.axis_index("c")
    sA, dA = eA.at[0], eA.at[1]
    sB, dB = eB.at[0], eB.at[1]
    CHG = CH * G

    def zero():
        def zcopy(k, _):
            pltpu.async_copy(zbuf, acc.at[pl.ds(s * RPT + k * ZR, ZR)], zsem)
            return 0
        lax.fori_loop(0, RPT // ZR, zcopy, 0)

        def zdrain(k, _):
            pltpu.make_async_copy(
                zbuf, acc.at[pl.ds(s * RPT + k * ZR, ZR)], zsem).wait()
            return 0
        lax.fori_loop(0, RPT // ZR, zdrain, 0)

    def edges(table, src_g, dst_g):
        def load_idx(t, sync):
            sl = lax.rem(t, 3)
            b = lax.rem(t, 2)
            e0 = (s * GPT + t * CH) * G
            if sync:
                pltpu.sync_copy(src_g.at[pl.ds(e0, CHG)], src_v.at[sl])
                pltpu.sync_copy(dst_g.at[pl.ds(e0, CHG)], dst_v.at[sl])
            else:
                pltpu.async_copy(src_g.at[pl.ds(e0, CHG)], src_v.at[sl],
                                 isem.at[b])
                pltpu.async_copy(dst_g.at[pl.ds(e0, CHG)], dst_v.at[sl],
                                 isem.at[b])

        def wait_idx(t):
            sl = lax.rem(t, 3)
            b = lax.rem(t, 2)
            e0 = (s * GPT + t * CH) * G
            pltpu.make_async_copy(src_g.at[pl.ds(e0, CHG)], src_v.at[sl],
                                  isem.at[b]).wait()
            pltpu.make_async_copy(dst_g.at[pl.ds(e0, CHG)], dst_v.at[sl],
                                  isem.at[b]).wait()

        def fire(t):
            sl = lax.rem(t, 3)
            b = lax.rem(t, 2)
            pltpu.async_copy(table.at[src_v.at[sl]], rows_v.at[b], gsem.at[b])

        def drain_g(t):
            sl = lax.rem(t, 3)
            b = lax.rem(t, 2)
            pltpu.make_async_copy(table.at[src_v.at[sl]], rows_v.at[b],
                                  gsem.at[b]).wait()

        def scat(t):
            sl = lax.rem(t, 3)
            b = lax.rem(t, 2)
            pltpu.async_copy(rows_v.at[b], acc.at[dst_v.at[sl]], ssem.at[b],
                             add=True)

        def drain_s(t):
            sl = lax.rem(t, 3)
            b = lax.rem(t, 2)
            pltpu.make_async_copy(rows_v.at[b], acc.at[dst_v.at[sl]],
                                  ssem.at[b]).wait()

        load_idx(0, True)
        fire(0)
        load_idx(1, True)

        def body(t, _):
            @pl.when(t >= 1)
            def _():
                drain_s(t - 1)

            @pl.when(t + 1 < NCHUNK)
            def _():
                @pl.when(t >= 1)
                def _():
                    wait_idx(t + 1)
                fire(t + 1)

            drain_g(t)
            scat(t)

            @pl.when(t + 2 < NCHUNK)
            def _():
                load_idx(t + 2, False)
            return 0

        lax.fori_loop(0, NCHUNK, body, 0)
        drain_s(NCHUNK - 1)

    def run(tA, tB, plane):
        def zfill(i, _):
            zbuf[i] = jnp.zeros((F,), jnp.float32)
            return 0
        lax.fori_loop(0, ZR, zfill, 0)

        zero()
        plsc.subcore_barrier()
        edges(tA, sA, dA)
        plsc.subcore_barrier()
        pltpu.sync_copy(acc.at[pl.ds(s * RPT, RPT)], outA.at[plane, s])
        zero()
        plsc.subcore_barrier()
        edges(tB, sB, dB)
        plsc.subcore_barrier()
        pltpu.sync_copy(acc.at[pl.ds(s * RPT, RPT)], outB.at[plane, s])

    @pl.when(c == 0)
    def _():
        run(tAlo, tBlo, 0)

    @pl.when(c == 1)
    def _():
        run(tAhi, tBhi, 1)


def _sc_mean_agg2(tAlo, tAhi, tBlo, tBhi, eA, eB):
    kern = pl.kernel(
        _agg_body,
        out_type=(jax.ShapeDtypeStruct((2, NT, RPT, F), jnp.float32),
                  jax.ShapeDtypeStruct((2, NT, RPT, F), jnp.float32)),
        mesh=plsc.VectorSubcoreMesh(core_axis_name="c", subcore_axis_name="s"),
        scratch_types=[
            pltpu.VMEM((3, CH * G), jnp.int32),
            pltpu.VMEM((3, CH * G), jnp.int32),
            pltpu.VMEM((2, CH * G, F), jnp.float32),
            pltpu.VMEM((ZR, F), jnp.float32),
            pltpu.VMEM_SHARED((N + 8, F), jnp.float32),
            pltpu.SemaphoreType.DMA((2,)),
            pltpu.SemaphoreType.DMA((2,)),
            pltpu.SemaphoreType.DMA((2,)),
            pltpu.SemaphoreType.DMA,
        ],
        compiler_params=pltpu.CompilerParams(use_tc_tiling_on_sc=False),
    )
    oA, oB = kern(tAlo, tAhi, tBlo, tBhi,
                  eA.reshape(2, NGP * G), eB.reshape(2, NGP * G))
    return oA.reshape(2, N, F), oB.reshape(2, N, F)


def _tc1_body(tr, msk, wt, wm, bias, av, h0o, m1lo, m1hi, m2lo, m2hi):
    a0 = av[0, 0]
    a11 = av[0, 1]
    a12 = av[0, 2]
    x = (jnp.dot(tr[...], wt[...], preferred_element_type=jnp.float32)
         + jnp.dot(msk[...], wm[...], preferred_element_type=jnp.float32)
         + bias[...])
    h0 = _prelu(x, a0)
    h0o[...] = h0
    m1 = _prelu(h0, a11)
    m2 = _prelu(h0, a12)
    lane = lax.broadcasted_iota(jnp.int32, (R, F), 1)
    one = lane == 14
    m1lo[...] = m1[:, :F]
    m1hi[...] = jnp.where(one, 1.0, m1[:, F:])
    m2lo[...] = m2[:, :F]
    m2hi[...] = jnp.where(one, 1.0, m2[:, F:])


def _tc2_body(h0, p1, p2, msk, w1h, w1pa, w1pb, w1m, b1,
              w2h, w2pa, w2pb, w2m, b2, A1a, A1b, ba1, A2a, A2b, ba2, av,
              h1ao, h1bo, g1lo, g1hi, g2lo, g2hi):
    a1 = av[0, 0]
    a21 = av[0, 1]
    a22 = av[0, 2]
    dot = functools.partial(jnp.dot, preferred_element_type=jnp.float32)
    h = h0[...]
    m = msk[...]
    p1a = p1[0]
    p1b = p1[1]
    p2a = p2[0]
    p2b = p2[1]
    c1 = jnp.maximum(p1b[:, 14:15], 1.0)
    c2 = jnp.maximum(p2b[:, 14:15], 1.0)
    t1 = (dot(h, w1h[...]) + dot(p1a / c1, w1pa[...]) + dot(p1b / c1, w1pb[...])
          + dot(m, w1m[...]) + b1[...])
    t2 = (dot(h, w2h[...]) + dot(p2a / c2, w2pa[...]) + dot(p2b / c2, w2pb[...])
          + dot(m, w2m[...]) + b2[...])
    h1a = _prelu(t1, a1)
    h1b = _prelu(t2, a1)
    h1ao[...] = h1a
    h1bo[...] = h1b
    g1 = _prelu(dot(h1a, A1a[...]) + dot(h1b, A1b[...]) + ba1[...], a21)
    g2 = _prelu(dot(h1a, A2a[...]) + dot(h1b, A2b[...]) + ba2[...], a22)
    lane = lax.broadcasted_iota(jnp.int32, (R, F), 1)
    one = lane == 14
    g1lo[...] = g1[:, :F]
    g1hi[...] = jnp.where(one, 1.0, g1[:, F:])
    g2lo[...] = g2[:, :F]
    g2hi[...] = jnp.where(one, 1.0, g2[:, F:])


def _tc3_body(h1a, h1b, q1, q2, msk, U1a, U1b, V1a, V1b, M1, c1b,
              U2a, U2b, V2a, V2b, M2, c2b, av, outo):
    a2 = av[0, 0]
    dot = functools.partial(jnp.dot, preferred_element_type=jnp.float32)
    ha = h1a[...]
    hb = h1b[...]
    m = msk[...]
    q1a = q1[0]
    q1b = q1[1]
    q2a = q2[0]
    q2b = q2[1]
    d1 = jnp.maximum(q1b[:, 14:15], 1.0)
    d2 = jnp.maximum(q2b[:, 14:15], 1.0)
    o1 = (dot(ha, U1a[...]) + dot(hb, U1b[...]) + dot(q1a / d1, V1a[...])
          + dot(q1b / d1, V1b[...]) + dot(m, M1[...]) + c1b[...])
    o2 = (dot(ha, U2a[...]) + dot(hb, U2b[...]) + dot(q2a / d2, V2a[...])
          + dot(q2b / d2, V2b[...]) + dot(m, M2[...]) + c2b[...])
    outo[...] = _prelu(jnp.concatenate([o1, o2], axis=1), a2)


def _full(shape):
    return pl.BlockSpec(shape, lambda i: tuple(0 for _ in shape))


def _rows(w):
    return pl.BlockSpec((R, w), lambda i: (i, 0))


def _plane(w):
    return pl.BlockSpec((2, R, w), lambda i: (0, i, 0))


def _pad(w, rpad, cpad):
    return jnp.pad(w, ((0, rpad), (0, cpad)))


def kernel(tr, mask, A_in_sta, A_in_src, W_init, b_init, W_l1t1, b_l1t1,
           W_l1t2, b_l1t2, W_l2t1a, b_l2t1a, W_l2t1b, b_l2t1b, W_l2t2a,
           b_l2t2a, W_l2t2b, b_l2t2b, a0, a11, a12, a1, a21, a22, a2):
    f32 = jnp.float32

    pad_blk = jnp.stack([jnp.zeros((NGP - NG0, G), jnp.int32),
                         jnp.full((NGP - NG0, G), N, jnp.int32)])

    def prep_edges(A):
        return jnp.concatenate(
            [A.astype(jnp.int32).reshape(2, NG0, G), pad_blk], axis=1)

    eA = prep_edges(A_in_sta)
    eB = prep_edges(A_in_src)

    wt = _pad(W_init[:128], 0, 2)
    wm = _pad(W_init[128:], 0, 2)
    bi = _pad(b_init[None, :], 0, 2)
    av1 = jnp.stack([a0, a11, a12, a0])[None, :]

    h0, m1lo, m1hi, m2lo, m2hi = pl.pallas_call(
        _tc1_body,
        grid=(N // R,),
        in_specs=[_rows(128), _rows(4), _full((128, 32)), _full((4, 32)),
                  _full((1, 32)), _full((1, 4))],
        out_specs=[_rows(32), _rows(F), _rows(F), _rows(F), _rows(F)],
        out_shape=[jax.ShapeDtypeStruct((N, 32), f32)]
        + [jax.ShapeDtypeStruct((N, F), f32)] * 4,
    )(tr, mask, wt, wm, bi, av1)

    p1, p2 = _sc_mean_agg2(m1lo, m1hi, m2lo, m2hi, eA, eB)

    def split1(Wf, bf):
        return (_pad(Wf[0:30], 2, 2), _pad(Wf[30:46], 0, 2),
                _pad(Wf[46:60], 2, 2), _pad(Wf[60:64], 0, 2),
                _pad(bf[None, :], 0, 2))
    w1h, w1pa, w1pb, w1m, b1 = split1(W_l1t1, b_l1t1)
    w2h, w2pa, w2pb, w2m, b2 = split1(W_l1t2, b_l1t2)
    A1a = _pad(W_l2t1a[0:30], 2, 2)
    A1b = _pad(W_l2t1a[30:60], 2, 2)
    ba1 = _pad(b_l2t1a[None, :], 0, 2)
    A2a = _pad(W_l2t2a[0:30], 2, 2)
    A2b = _pad(W_l2t2a[30:60], 2, 2)
    ba2 = _pad(b_l2t2a[None, :], 0, 2)
    av2 = jnp.stack([a1, a21, a22, a1])[None, :]

    h1a, h1b, g1lo, g1hi, g2lo, g2hi = pl.pallas_call(
        _tc2_body,
        grid=(N // R,),
        in_specs=[_rows(32), _plane(F), _plane(F), _rows(4),
                  _full((32, 32)), _full((16, 32)), _full((16, 32)),
                  _full((4, 32)), _full((1, 32)),
                  _full((32, 32)), _full((16, 32)), _full((16, 32)),
                  _full((4, 32)), _full((1, 32)),
                  _full((32, 32)), _full((32, 32)), _full((1, 32)),
                  _full((32, 32)), _full((32, 32)), _full((1, 32)),
                  _full((1, 4))],
        out_specs=[_rows(32), _rows(32), _rows(F), _rows(F), _rows(F),
                   _rows(F)],
        out_shape=[jax.ShapeDtypeStruct((N, 32), f32)] * 2
        + [jax.ShapeDtypeStruct((N, F), f32)] * 4,
    )(h0, p1, p2, mask, w1h, w1pa, w1pb, w1m, b1, w2h, w2pa, w2pb, w2m, b2,
      A1a, A1b, ba1, A2a, A2b, ba2, av2)

    q1, q2 = _sc_mean_agg2(g1lo, g1hi, g2lo, g2hi, eA, eB)

    def split3(Wf, bf):
        return (_pad(Wf[0:30], 2, 0), _pad(Wf[30:60], 2, 0), Wf[60:76],
                _pad(Wf[76:90], 2, 0), Wf[90:94], bf[None, :])
    U1a, U1b, V1a, V1b, M1, c1b = split3(W_l2t1b, b_l2t1b)
    U2a, U2b, V2a, V2b, M2, c2b = split3(W_l2t2b, b_l2t2b)
    av3 = jnp.stack([a2, a2, a2, a2])[None, :]

    out = pl.pallas_call(
        _tc3_body,
        grid=(N // R,),
        in_specs=[_rows(32), _rows(32), _plane(F), _plane(F), _rows(4),
                  _full((32, 128)), _full((32, 128)), _full((16, 128)),
                  _full((16, 128)), _full((4, 128)), _full((1, 128)),
                  _full((32, 128)), _full((32, 128)), _full((16, 128)),
                  _full((16, 128)), _full((4, 128)), _full((1, 128)),
                  _full((1, 4))],
        out_specs=[_rows(256)],
        out_shape=[jax.ShapeDtypeStruct((N, 256), f32)],
    )(h1a, h1b, q1, q2, mask, U1a, U1b, V1a, V1b, M1, c1b,
      U2a, U2b, V2a, V2b, M2, c2b, av3)

    return out[0]


# speedup vs baseline: 15.5390x; 1.0089x over previous
import functools

import jax
import jax.numpy as jnp
from jax import lax
from jax.experimental import pallas as pl
from jax.experimental.pallas import tpu as pltpu
from jax.experimental.pallas import tpu_sc as plsc

N = 100000
E = 1600000
F = 16
G = 128
CH = 4
NT = 16
GPT = 784
NCHUNK = GPT // CH
NG0 = E // G
NGP = NT * GPT
PK = 8
RPT = N // NT
ZR = 125
R = 4000


def _prelu(x, a):
    return jnp.where(x >= 0, x, a * x)


def _agg_body(tAlo, tAhi, tBlo, tBhi, eA, eB, outA, outB,
              src_v, dst_v, rows_v, zbuf, acc, gsem, ssem, isem, zsem):
    s = lax.axis_index("s")
    c = lax.axis_index("c")
    sA, dA = eA.at[0], eA.at[1]
    sB, dB = eB.at[0], eB.at[1]
    CHG = CH * G

    def zero():
        def zcopy(k, _):
            pltpu.async_copy(zbuf, acc.at[pl.ds(s * RPT + k * ZR, ZR)], zsem)
            return 0
        lax.fori_loop(0, RPT // ZR, zcopy, 0)

        def zdrain(k, _):
            pltpu.make_async_copy(
                zbuf, acc.at[pl.ds(s * RPT + k * ZR, ZR)], zsem).wait()
            return 0
        lax.fori_loop(0, RPT // ZR, zdrain, 0)

    def edges(table, src_g, dst_g):
        def load_idx(t, sync):
            sl = lax.rem(t, 3)
            b = lax.rem(t, 2)
            e0 = (s * GPT + t * CH) * G
            if sync:
                pltpu.sync_copy(src_g.at[pl.ds(e0, CHG)], src_v.at[sl])
                pltpu.sync_copy(dst_g.at[pl.ds(e0, CHG)], dst_v.at[sl])
            else:
                pltpu.async_copy(src_g.at[pl.ds(e0, CHG)], src_v.at[sl],
                                 isem.at[b])
                pltpu.async_copy(dst_g.at[pl.ds(e0, CHG)], dst_v.at[sl],
                                 isem.at[b])

        def wait_idx(t):
            sl = lax.rem(t, 3)
            b = lax.rem(t, 2)
            e0 = (s * GPT + t * CH) * G
            pltpu.make_async_copy(src_g.at[pl.ds(e0, CHG)], src_v.at[sl],
                                  isem.at[b]).wait()
            pltpu.make_async_copy(dst_g.at[pl.ds(e0, CHG)], dst_v.at[sl],
                                  isem.at[b]).wait()

        def fire(t):
            sl = lax.rem(t, 3)
            b = lax.rem(t, 2)
            pltpu.async_copy(table.at[src_v.at[sl]], rows_v.at[b], gsem.at[b])

        def drain_g(t):
            sl = lax.rem(t, 3)
            b = lax.rem(t, 2)
            pltpu.make_async_copy(table.at[src_v.at[sl]], rows_v.at[b],
                                  gsem.at[b]).wait()

        def scat(t):
            sl = lax.rem(t, 3)
            b = lax.rem(t, 2)
            pltpu.async_copy(rows_v.at[b], acc.at[dst_v.at[sl]], ssem.at[b],
                             add=True)

        def drain_s(t):
            sl = lax.rem(t, 3)
            b = lax.rem(t, 2)
            pltpu.make_async_copy(rows_v.at[b], acc.at[dst_v.at[sl]],
                                  ssem.at[b]).wait()

        load_idx(0, True)
        fire(0)
        load_idx(1, True)

        def body(t, _):
            @pl.when(t >= 1)
            def _():
                drain_s(t - 1)

            @pl.when(t + 1 < NCHUNK)
            def _():
                @pl.when(t >= 1)
                def _():
                    wait_idx(t + 1)
                fire(t + 1)

            drain_g(t)
            scat(t)

            @pl.when(t + 2 < NCHUNK)
            def _():
                load_idx(t + 2, False)
            return 0

        lax.fori_loop(0, NCHUNK, body, 0)
        drain_s(NCHUNK - 1)

    def run(tA, tB, plane):
        def zfill(i, _):
            zbuf[i] = jnp.zeros((F,), jnp.float32)
            return 0
        lax.fori_loop(0, ZR, zfill, 0)

        zero()
        plsc.subcore_barrier()
        edges(tA, sA, dA)
        plsc.subcore_barrier()
        pltpu.sync_copy(acc.at[pl.ds(s * RPT, RPT)], outA.at[plane, s])
        zero()
        plsc.subcore_barrier()
        edges(tB, sB, dB)
        plsc.subcore_barrier()
        pltpu.sync_copy(acc.at[pl.ds(s * RPT, RPT)], outB.at[plane, s])

    @pl.when(c == 0)
    def _():
        run(tAlo, tBlo, 0)

    @pl.when(c == 1)
    def _():
        run(tAhi, tBhi, 1)


def _sc_mean_agg2(tAlo, tAhi, tBlo, tBhi, eA, eB):
    kern = pl.kernel(
        _agg_body,
        out_type=(jax.ShapeDtypeStruct((2, NT, RPT, F), jnp.float32),
                  jax.ShapeDtypeStruct((2, NT, RPT, F), jnp.float32)),
        mesh=plsc.VectorSubcoreMesh(core_axis_name="c", subcore_axis_name="s"),
        scratch_types=[
            pltpu.VMEM((3, CH * G), jnp.int32),
            pltpu.VMEM((3, CH * G), jnp.int32),
            pltpu.VMEM((2, CH * G, F), jnp.float32),
            pltpu.VMEM((ZR, F), jnp.float32),
            pltpu.VMEM_SHARED((N + 8, F), jnp.float32),
            pltpu.SemaphoreType.DMA((2,)),
            pltpu.SemaphoreType.DMA((2,)),
            pltpu.SemaphoreType.DMA((2,)),
            pltpu.SemaphoreType.DMA,
        ],
        compiler_params=pltpu.CompilerParams(use_tc_tiling_on_sc=False),
    )
    oA, oB = kern(tAlo, tAhi, tBlo, tBhi,
                  eA.reshape(2, NGP * G), eB.reshape(2, NGP * G))
    return oA.reshape(2, N, F), oB.reshape(2, N, F)


def _tc1_body(tr, msk, wt, wm, bias, av, h0o, m1lo, m1hi, m2lo, m2hi):
    a0 = av[0, 0]
    a11 = av[0, 1]
    a12 = av[0, 2]
    x = (jnp.dot(tr[...], wt[...], preferred_element_type=jnp.float32)
         + jnp.dot(msk[...], wm[...], preferred_element_type=jnp.float32)
         + bias[...])
    h0 = _prelu(x, a0)
    h0o[...] = h0
    m1 = _prelu(h0, a11)
    m2 = _prelu(h0, a12)
    lane = lax.broadcasted_iota(jnp.int32, (R, F), 1)
    one = lane == 14
    m1lo[...] = m1[:, :F]
    m1hi[...] = jnp.where(one, 1.0, m1[:, F:])
    m2lo[...] = m2[:, :F]
    m2hi[...] = jnp.where(one, 1.0, m2[:, F:])


def _tc2_body(h0, p1, p2, msk, w1h, w1pa, w1pb, w1m, b1,
              w2h, w2pa, w2pb, w2m, b2, A1a, A1b, ba1, A2a, A2b, ba2, av,
              h1ao, h1bo, g1lo, g1hi, g2lo, g2hi):
    a1 = av[0, 0]
    a21 = av[0, 1]
    a22 = av[0, 2]
    dot = functools.partial(jnp.dot, preferred_element_type=jnp.float32)
    h = h0[...]
    m = msk[...]
    p1a = p1[0]
    p1b = p1[1]
    p2a = p2[0]
    p2b = p2[1]
    c1 = jnp.maximum(p1b[:, 14:15], 1.0)
    c2 = jnp.maximum(p2b[:, 14:15], 1.0)
    t1 = (dot(h, w1h[...]) + dot(p1a / c1, w1pa[...]) + dot(p1b / c1, w1pb[...])
          + dot(m, w1m[...]) + b1[...])
    t2 = (dot(h, w2h[...]) + dot(p2a / c2, w2pa[...]) + dot(p2b / c2, w2pb[...])
          + dot(m, w2m[...]) + b2[...])
    h1a = _prelu(t1, a1)
    h1b = _prelu(t2, a1)
    h1ao[...] = h1a
    h1bo[...] = h1b
    g1 = _prelu(dot(h1a, A1a[...]) + dot(h1b, A1b[...]) + ba1[...], a21)
    g2 = _prelu(dot(h1a, A2a[...]) + dot(h1b, A2b[...]) + ba2[...], a22)
    lane = lax.broadcasted_iota(jnp.int32, (R, F), 1)
    one = lane == 14
    g1lo[...] = g1[:, :F]
    g1hi[...] = jnp.where(one, 1.0, g1[:, F:])
    g2lo[...] = g2[:, :F]
    g2hi[...] = jnp.where(one, 1.0, g2[:, F:])


def _tc3_body(h1a, h1b, q1, q2, msk, U1a, U1b, V1a, V1b, M1, c1b,
              U2a, U2b, V2a, V2b, M2, c2b, av, outo):
    a2 = av[0, 0]
    dot = functools.partial(jnp.dot, preferred_element_type=jnp.float32)
    ha = h1a[...]
    hb = h1b[...]
    m = msk[...]
    q1a = q1[0]
    q1b = q1[1]
    q2a = q2[0]
    q2b = q2[1]
    d1 = jnp.maximum(q1b[:, 14:15], 1.0)
    d2 = jnp.maximum(q2b[:, 14:15], 1.0)
    o1 = (dot(ha, U1a[...]) + dot(hb, U1b[...]) + dot(q1a / d1, V1a[...])
          + dot(q1b / d1, V1b[...]) + dot(m, M1[...]) + c1b[...])
    o2 = (dot(ha, U2a[...]) + dot(hb, U2b[...]) + dot(q2a / d2, V2a[...])
          + dot(q2b / d2, V2b[...]) + dot(m, M2[...]) + c2b[...])
    outo[...] = _prelu(jnp.concatenate([o1, o2], axis=1), a2)


def _full(shape):
    return pl.BlockSpec(shape, lambda i: tuple(0 for _ in shape))


def _rows(w):
    return pl.BlockSpec((R, w), lambda i: (i, 0))


def _plane(w):
    return pl.BlockSpec((2, R, w), lambda i: (0, i, 0))


def _pad(w, rpad, cpad):
    return jnp.pad(w, ((0, rpad), (0, cpad)))


def kernel(tr, mask, A_in_sta, A_in_src, W_init, b_init, W_l1t1, b_l1t1,
           W_l1t2, b_l1t2, W_l2t1a, b_l2t1a, W_l2t1b, b_l2t1b, W_l2t2a,
           b_l2t2a, W_l2t2b, b_l2t2b, a0, a11, a12, a1, a21, a22, a2):
    f32 = jnp.float32

    pad_blk = jnp.stack([jnp.zeros((NGP - NG0, G), jnp.int32),
                         jnp.full((NGP - NG0, G), N, jnp.int32)])

    def prep_edges(A):
        return jnp.concatenate(
            [A.astype(jnp.int32).reshape(2, NG0, G), pad_blk], axis=1)

    eA = prep_edges(A_in_sta)
    eB = prep_edges(A_in_src)

    wt = _pad(W_init[:128], 0, 2)
    wm = _pad(W_init[128:], 0, 2)
    bi = _pad(b_init[None, :], 0, 2)
    av1 = jnp.stack([a0, a11, a12, a0])[None, :]

    h0, m1lo, m1hi, m2lo, m2hi = pl.pallas_call(
        _tc1_body,
        grid=(N // R,),
        in_specs=[_rows(128), _rows(4), _full((128, 32)), _full((4, 32)),
                  _full((1, 32)), _full((1, 4))],
        out_specs=[_rows(32), _rows(F), _rows(F), _rows(F), _rows(F)],
        out_shape=[jax.ShapeDtypeStruct((N, 32), f32)]
        + [jax.ShapeDtypeStruct((N, F), f32)] * 4,
    )(tr, mask, wt, wm, bi, av1)

    p1, p2 = _sc_mean_agg2(m1lo, m1hi, m2lo, m2hi, eA, eB)

    def split1(Wf, bf):
        return (_pad(Wf[0:30], 2, 2), _pad(Wf[30:46], 0, 2),
                _pad(Wf[46:60], 2, 2), _pad(Wf[60:64], 0, 2),
                _pad(bf[None, :], 0, 2))
    w1h, w1pa, w1pb, w1m, b1 = split1(W_l1t1, b_l1t1)
    w2h, w2pa, w2pb, w2m, b2 = split1(W_l1t2, b_l1t2)
    A1a = _pad(W_l2t1a[0:30], 2, 2)
    A1b = _pad(W_l2t1a[30:60], 2, 2)
    ba1 = _pad(b_l2t1a[None, :], 0, 2)
    A2a = _pad(W_l2t2a[0:30], 2, 2)
    A2b = _pad(W_l2t2a[30:60], 2, 2)
    ba2 = _pad(b_l2t2a[None, :], 0, 2)
    av2 = jnp.stack([a1, a21, a22, a1])[None, :]

    h1a, h1b, g1lo, g1hi, g2lo, g2hi = pl.pallas_call(
        _tc2_body,
        grid=(N // R,),
        in_specs=[_rows(32), _plane(F), _plane(F), _rows(4),
                  _full((32, 32)), _full((16, 32)), _full((16, 32)),
                  _full((4, 32)), _full((1, 32)),
                  _full((32, 32)), _full((16, 32)), _full((16, 32)),
                  _full((4, 32)), _full((1, 32)),
                  _full((32, 32)), _full((32, 32)), _full((1, 32)),
                  _full((32, 32)), _full((32, 32)), _full((1, 32)),
                  _full((1, 4))],
        out_specs=[_rows(32), _rows(32), _rows(F), _rows(F), _rows(F),
                   _rows(F)],
        out_shape=[jax.ShapeDtypeStruct((N, 32), f32)] * 2
        + [jax.ShapeDtypeStruct((N, F), f32)] * 4,
    )(h0, p1, p2, mask, w1h, w1pa, w1pb, w1m, b1, w2h, w2pa, w2pb, w2m, b2,
      A1a, A1b, ba1, A2a, A2b, ba2, av2)

    q1, q2 = _sc_mean_agg2(g1lo, g1hi, g2lo, g2hi, eA, eB)

    def split3(Wf, bf):
        return (_pad(Wf[0:30], 2, 0), _pad(Wf[30:60], 2, 0), Wf[60:76],
                _pad(Wf[76:90], 2, 0), Wf[90:94], bf[None, :])
    U1a, U1b, V1a, V1b, M1, c1b = split3(W_l2t1b, b_l2t1b)
    U2a, U2b, V2a, V2b, M2, c2b = split3(W_l2t2b, b_l2t2b)
    av3 = jnp.stack([a2, a2, a2, a2])[None, :]

    out = pl.pallas_call(
        _tc3_body,
        grid=(N // R,),
        in_specs=[_rows(32), _rows(32), _plane(F), _plane(F), _rows(4),
                  _full((32, 128)), _full((32, 128)), _full((16, 128)),
                  _full((16, 128)), _full((4, 128)), _full((1, 128)),
                  _full((32, 128)), _full((32, 128)), _full((16, 128)),
                  _full((16, 128)), _full((4, 128)), _full((1, 128)),
                  _full((1, 4))],
        out_specs=[_rows(256)],
        out_shape=[jax.ShapeDtypeStruct((N, 256), f32)],
    )(h1a, h1b, q1, q2, mask, U1a, U1b, V1a, V1b, M1, c1b,
      U2a, U2b, V2a, V2b, M2, c2b, av3)

    return out[0]


# speedup vs baseline: 15.8843x; 1.0222x over previous
import functools

import jax
import jax.numpy as jnp
from jax import lax
from jax.experimental import pallas as pl
from jax.experimental.pallas import tpu as pltpu
from jax.experimental.pallas import tpu_sc as plsc

N = 100000
E = 1600000
F = 16
G = 128
CH = 4
NT = 16
GPT = 784
NCHUNK = GPT // CH
NG0 = E // G
NGP = NT * GPT
PK = 8
RPT = N // NT
ZR = 125
R = 4000


def _prelu(x, a):
    return jnp.where(x >= 0, x, a * x)


def _agg_body(tAlo, tAhi, tBlo, tBhi, eA, eB, outA, outB,
              src_v, dst_v, rows_v, zbuf, acc, gsem, ssem, isem, zsem):
    s = lax.axis_index("s")
    c = lax.axis_index("c")
    sA, dA = eA.at[0], eA.at[1]
    sB, dB = eB.at[0], eB.at[1]
    CHG = CH * G

    def zero():
        def zcopy(k, _):
            pltpu.async_copy(zbuf, acc.at[pl.ds(s * RPT + k * ZR, ZR)], zsem)
            return 0
        lax.fori_loop(0, RPT // ZR, zcopy, 0)

        def zdrain(k, _):
            pltpu.make_async_copy(
                zbuf, acc.at[pl.ds(s * RPT + k * ZR, ZR)], zsem).wait()
            return 0
        lax.fori_loop(0, RPT // ZR, zdrain, 0)

    def edges(table, src_g, dst_g):
        def load_idx(t, sync):
            sl = lax.rem(t, 3)
            b = lax.rem(t, 2)
            e0 = (s * GPT + t * CH) * G
            if sync:
                pltpu.sync_copy(src_g.at[pl.ds(e0, CHG)], src_v.at[sl])
                pltpu.sync_copy(dst_g.at[pl.ds(e0, CHG)], dst_v.at[sl])
            else:
                pltpu.async_copy(src_g.at[pl.ds(e0, CHG)], src_v.at[sl],
                                 isem.at[b])
                pltpu.async_copy(dst_g.at[pl.ds(e0, CHG)], dst_v.at[sl],
                                 isem.at[b])

        def wait_idx(t):
            sl = lax.rem(t, 3)
            b = lax.rem(t, 2)
            e0 = (s * GPT + t * CH) * G
            pltpu.make_async_copy(src_g.at[pl.ds(e0, CHG)], src_v.at[sl],
                                  isem.at[b]).wait()
            pltpu.make_async_copy(dst_g.at[pl.ds(e0, CHG)], dst_v.at[sl],
                                  isem.at[b]).wait()

        def fire(t):
            sl = lax.rem(t, 3)
            b = lax.rem(t, 2)
            pltpu.async_copy(table.at[src_v.at[sl]], rows_v.at[b], gsem.at[b])

        def drain_g(t):
            sl = lax.rem(t, 3)
            b = lax.rem(t, 2)
            pltpu.make_async_copy(table.at[src_v.at[sl]], rows_v.at[b],
                                  gsem.at[b]).wait()

        def scat(t):
            sl = lax.rem(t, 3)
            b = lax.rem(t, 2)
            pltpu.async_copy(rows_v.at[b], acc.at[dst_v.at[sl]], ssem.at[b],
                             add=True)

        def drain_s(t):
            sl = lax.rem(t, 3)
            b = lax.rem(t, 2)
            pltpu.make_async_copy(rows_v.at[b], acc.at[dst_v.at[sl]],
                                  ssem.at[b]).wait()

        load_idx(0, True)
        fire(0)
        load_idx(1, True)

        def body(t, _):
            @pl.when(t >= 1)
            def _():
                drain_s(t - 1)

            @pl.when(t + 1 < NCHUNK)
            def _():
                @pl.when(t >= 1)
                def _():
                    wait_idx(t + 1)
                fire(t + 1)

            drain_g(t)
            scat(t)

            @pl.when(t + 2 < NCHUNK)
            def _():
                load_idx(t + 2, False)
            return 0

        lax.fori_loop(0, NCHUNK, body, 0)
        drain_s(NCHUNK - 1)

    def run(tA, tB, plane):
        def zfill(i, _):
            zbuf[i] = jnp.zeros((F,), jnp.float32)
            return 0
        lax.fori_loop(0, ZR, zfill, 0)

        zero()
        plsc.subcore_barrier()
        edges(tA, sA, dA)
        plsc.subcore_barrier()
        pltpu.sync_copy(acc.at[pl.ds(s * RPT, RPT)],
                        outA.at[plane, pl.ds(s * RPT, RPT)])
        zero()
        plsc.subcore_barrier()
        edges(tB, sB, dB)
        plsc.subcore_barrier()
        pltpu.sync_copy(acc.at[pl.ds(s * RPT, RPT)],
                        outB.at[plane, pl.ds(s * RPT, RPT)])

    @pl.when(c == 0)
    def _():
        run(tAlo, tBlo, 0)

    @pl.when(c == 1)
    def _():
        run(tAhi, tBhi, 1)


def _sc_mean_agg2(tAlo, tAhi, tBlo, tBhi, eA, eB):
    kern = pl.kernel(
        _agg_body,
        out_type=(jax.ShapeDtypeStruct((2, N, F), jnp.float32),
                  jax.ShapeDtypeStruct((2, N, F), jnp.float32)),
        mesh=plsc.VectorSubcoreMesh(core_axis_name="c", subcore_axis_name="s"),
        scratch_types=[
            pltpu.VMEM((3, CH * G), jnp.int32),
            pltpu.VMEM((3, CH * G), jnp.int32),
            pltpu.VMEM((2, CH * G, F), jnp.float32),
            pltpu.VMEM((ZR, F), jnp.float32),
            pltpu.VMEM_SHARED((N + 8, F), jnp.float32),
            pltpu.SemaphoreType.DMA((2,)),
            pltpu.SemaphoreType.DMA((2,)),
            pltpu.SemaphoreType.DMA((2,)),
            pltpu.SemaphoreType.DMA,
        ],
        compiler_params=pltpu.CompilerParams(use_tc_tiling_on_sc=False),
    )
    return kern(tAlo, tAhi, tBlo, tBhi, eA, eB)


def _tc1_body(tr, msk, wt, wm, bias, av, h0o, m1lo, m1hi, m2lo, m2hi):
    a0 = av[0, 0]
    a11 = av[0, 1]
    a12 = av[0, 2]
    x = (jnp.dot(tr[...], wt[...], preferred_element_type=jnp.float32)
         + jnp.dot(msk[...], wm[...], preferred_element_type=jnp.float32)
         + bias[...])
    h0 = _prelu(x, a0)
    h0o[...] = h0
    m1 = _prelu(h0, a11)
    m2 = _prelu(h0, a12)
    lane = lax.broadcasted_iota(jnp.int32, (R, F), 1)
    one = lane == 14
    m1lo[...] = m1[:, :F]
    m1hi[...] = jnp.where(one, 1.0, m1[:, F:])
    m2lo[...] = m2[:, :F]
    m2hi[...] = jnp.where(one, 1.0, m2[:, F:])


def _tc2_body(h0, p1, p2, msk, w1h, w1pa, w1pb, w1m, b1,
              w2h, w2pa, w2pb, w2m, b2, A1a, A1b, ba1, A2a, A2b, ba2, av,
              h1ao, h1bo, g1lo, g1hi, g2lo, g2hi):
    a1 = av[0, 0]
    a21 = av[0, 1]
    a22 = av[0, 2]
    dot = functools.partial(jnp.dot, preferred_element_type=jnp.float32)
    h = h0[...]
    m = msk[...]
    p1a = p1[0]
    p1b = p1[1]
    p2a = p2[0]
    p2b = p2[1]
    c1 = jnp.maximum(p1b[:, 14:15], 1.0)
    c2 = jnp.maximum(p2b[:, 14:15], 1.0)
    t1 = (dot(h, w1h[...]) + dot(p1a / c1, w1pa[...]) + dot(p1b / c1, w1pb[...])
          + dot(m, w1m[...]) + b1[...])
    t2 = (dot(h, w2h[...]) + dot(p2a / c2, w2pa[...]) + dot(p2b / c2, w2pb[...])
          + dot(m, w2m[...]) + b2[...])
    h1a = _prelu(t1, a1)
    h1b = _prelu(t2, a1)
    h1ao[...] = h1a
    h1bo[...] = h1b
    g1 = _prelu(dot(h1a, A1a[...]) + dot(h1b, A1b[...]) + ba1[...], a21)
    g2 = _prelu(dot(h1a, A2a[...]) + dot(h1b, A2b[...]) + ba2[...], a22)
    lane = lax.broadcasted_iota(jnp.int32, (R, F), 1)
    one = lane == 14
    g1lo[...] = g1[:, :F]
    g1hi[...] = jnp.where(one, 1.0, g1[:, F:])
    g2lo[...] = g2[:, :F]
    g2hi[...] = jnp.where(one, 1.0, g2[:, F:])


def _tc3_body(h1a, h1b, q1, q2, msk, U1a, U1b, V1a, V1b, M1, c1b,
              U2a, U2b, V2a, V2b, M2, c2b, av, outo):
    a2 = av[0, 0]
    dot = functools.partial(jnp.dot, preferred_element_type=jnp.float32)
    ha = h1a[...]
    hb = h1b[...]
    m = msk[...]
    q1a = q1[0]
    q1b = q1[1]
    q2a = q2[0]
    q2b = q2[1]
    d1 = jnp.maximum(q1b[:, 14:15], 1.0)
    d2 = jnp.maximum(q2b[:, 14:15], 1.0)
    o1 = (dot(ha, U1a[...]) + dot(hb, U1b[...]) + dot(q1a / d1, V1a[...])
          + dot(q1b / d1, V1b[...]) + dot(m, M1[...]) + c1b[...])
    o2 = (dot(ha, U2a[...]) + dot(hb, U2b[...]) + dot(q2a / d2, V2a[...])
          + dot(q2b / d2, V2b[...]) + dot(m, M2[...]) + c2b[...])
    outo[...] = _prelu(jnp.concatenate([o1, o2], axis=1), a2)


def _full(shape):
    return pl.BlockSpec(shape, lambda i: tuple(0 for _ in shape))


def _rows(w):
    return pl.BlockSpec((R, w), lambda i: (i, 0))


def _plane(w):
    return pl.BlockSpec((2, R, w), lambda i: (0, i, 0))


def _pad(w, rpad, cpad):
    return jnp.pad(w, ((0, rpad), (0, cpad)))


def kernel(tr, mask, A_in_sta, A_in_src, W_init, b_init, W_l1t1, b_l1t1,
           W_l1t2, b_l1t2, W_l2t1a, b_l2t1a, W_l2t1b, b_l2t1b, W_l2t2a,
           b_l2t2a, W_l2t2b, b_l2t2b, a0, a11, a12, a1, a21, a22, a2):
    f32 = jnp.float32

    def prep_edges(A):
        return jnp.pad(A.astype(jnp.int32), ((0, 0), (0, NGP * G - E)),
                       constant_values=N)

    eA = prep_edges(A_in_sta)
    eB = prep_edges(A_in_src)

    wt = _pad(W_init[:128], 0, 2)
    wm = _pad(W_init[128:], 0, 2)
    bi = _pad(b_init[None, :], 0, 2)
    av1 = jnp.stack([a0, a11, a12, a0])[None, :]

    h0, m1lo, m1hi, m2lo, m2hi = pl.pallas_call(
        _tc1_body,
        grid=(N // R,),
        in_specs=[_rows(128), _rows(4), _full((128, 32)), _full((4, 32)),
                  _full((1, 32)), _full((1, 4))],
        out_specs=[_rows(32), _rows(F), _rows(F), _rows(F), _rows(F)],
        out_shape=[jax.ShapeDtypeStruct((N, 32), f32)]
        + [jax.ShapeDtypeStruct((N + 8, F), f32)] * 4,
    )(tr, mask, wt, wm, bi, av1)

    p1, p2 = _sc_mean_agg2(m1lo, m1hi, m2lo, m2hi, eA, eB)

    def split1(Wf, bf):
        return (_pad(Wf[0:30], 2, 2), _pad(Wf[30:46], 0, 2),
                _pad(Wf[46:60], 2, 2), _pad(Wf[60:64], 0, 2),
                _pad(bf[None, :], 0, 2))
    w1h, w1pa, w1pb, w1m, b1 = split1(W_l1t1, b_l1t1)
    w2h, w2pa, w2pb, w2m, b2 = split1(W_l1t2, b_l1t2)
    A1a = _pad(W_l2t1a[0:30], 2, 2)
    A1b = _pad(W_l2t1a[30:60], 2, 2)
    ba1 = _pad(b_l2t1a[None, :], 0, 2)
    A2a = _pad(W_l2t2a[0:30], 2, 2)
    A2b = _pad(W_l2t2a[30:60], 2, 2)
    ba2 = _pad(b_l2t2a[None, :], 0, 2)
    av2 = jnp.stack([a1, a21, a22, a1])[None, :]

    h1a, h1b, g1lo, g1hi, g2lo, g2hi = pl.pallas_call(
        _tc2_body,
        grid=(N // R,),
        in_specs=[_rows(32), _plane(F), _plane(F), _rows(4),
                  _full((32, 32)), _full((16, 32)), _full((16, 32)),
                  _full((4, 32)), _full((1, 32)),
                  _full((32, 32)), _full((16, 32)), _full((16, 32)),
                  _full((4, 32)), _full((1, 32)),
                  _full((32, 32)), _full((32, 32)), _full((1, 32)),
                  _full((32, 32)), _full((32, 32)), _full((1, 32)),
                  _full((1, 4))],
        out_specs=[_rows(32), _rows(32), _rows(F), _rows(F), _rows(F),
                   _rows(F)],
        out_shape=[jax.ShapeDtypeStruct((N, 32), f32)] * 2
        + [jax.ShapeDtypeStruct((N + 8, F), f32)] * 4,
    )(h0, p1, p2, mask, w1h, w1pa, w1pb, w1m, b1, w2h, w2pa, w2pb, w2m, b2,
      A1a, A1b, ba1, A2a, A2b, ba2, av2)

    q1, q2 = _sc_mean_agg2(g1lo, g1hi, g2lo, g2hi, eA, eB)

    def split3(Wf, bf):
        return (_pad(Wf[0:30], 2, 0), _pad(Wf[30:60], 2, 0), Wf[60:76],
                _pad(Wf[76:90], 2, 0), Wf[90:94], bf[None, :])
    U1a, U1b, V1a, V1b, M1, c1b = split3(W_l2t1b, b_l2t1b)
    U2a, U2b, V2a, V2b, M2, c2b = split3(W_l2t2b, b_l2t2b)
    av3 = jnp.stack([a2, a2, a2, a2])[None, :]

    out = pl.pallas_call(
        _tc3_body,
        grid=(N // R,),
        in_specs=[_rows(32), _rows(32), _plane(F), _plane(F), _rows(4),
                  _full((32, 128)), _full((32, 128)), _full((16, 128)),
                  _full((16, 128)), _full((4, 128)), _full((1, 128)),
                  _full((32, 128)), _full((32, 128)), _full((16, 128)),
                  _full((16, 128)), _full((4, 128)), _full((1, 128)),
                  _full((1, 4))],
        out_specs=[_rows(256)],
        out_shape=[jax.ShapeDtypeStruct((N, 256), f32)],
    )(h1a, h1b, q1, q2, mask, U1a, U1b, V1a, V1b, M1, c1b,
      U2a, U2b, V2a, V2b, M2, c2b, av3)

    return out[0]
